# baseline (speedup 1.0000x reference)
import functools

import jax
import jax.numpy as jnp
from jax import lax
from jax.experimental import pallas as pl
from jax.experimental.pallas import tpu as pltpu
from jax.experimental.pallas import tpu_sc as plsc

N = 10000
E = 160000
D = 128
NBOND = 4

NC = 2
NS = 16
NW = NC * NS

C = 40
EPT_ALL = E // NW
EPT_CORE = E // NS
NP = 10240
RPT = NP // NS
GPT = 320
NPAD = GPT * NW

_mesh = functools.partial(
    plsc.VectorSubcoreMesh, core_axis_name="c", subcore_axis_name="s",
    num_cores=NC, num_subcores=NS)


def _sigmoid(x):
    return 1.0 / (1.0 + jnp.exp(-x))


def _mm(x, w_ref, b_ref):
    return lax.dot_general(x, w_ref[...], (((1,), (1,)), ((), ())),
                           preferred_element_type=jnp.float32) + b_ref[...]


def _matvec_body(wf_ref, f_ref, bf_ref, o_ref):
    r = lax.dot_general(
        wf_ref[...], f_ref[...], (((1,), (1,)), ((), ())),
        preferred_element_type=jnp.float32)
    v = r[:, 0:1] + bf_ref[:, 0:1]
    iv = jnp.clip(jnp.abs(v.astype(jnp.int32)), 0, N - 1)
    o_ref[...] = jnp.broadcast_to(iv, (400, 128))


def _k1_findex(Wf, f, bf):
    f2 = jnp.broadcast_to(f[None, :], (8, N))
    bf2 = jnp.broadcast_to(bf[:, None], (N, 128))
    out = pl.pallas_call(
        _matvec_body,
        grid=(25,),
        in_specs=[
            pl.BlockSpec((400, N), lambda i: (i, 0)),
            pl.BlockSpec((8, N), lambda i: (0, 0)),
            pl.BlockSpec((400, 128), lambda i: (i, 0)),
        ],
        out_specs=pl.BlockSpec((400, 128), lambda i: (i, 0)),
        out_shape=jax.ShapeDtypeStruct((N, 128), jnp.int32),
    )(Wf, f2, bf2)
    return out[:, 0]


def _k2_gather_body(ne_hbm, hidx_hbm, pe_hbm, fi_hbm, outh, outp,
                    idx_v, rows_v, sem):
    wid = lax.axis_index("s") * NC + lax.axis_index("c")
    base = wid * GPT
    pltpu.sync_copy(hidx_hbm.at[pl.ds(base, GPT)], idx_v)
    pltpu.async_copy(ne_hbm.at[idx_v], rows_v, sem).wait()
    pltpu.sync_copy(rows_v, outh.at[pl.ds(base, GPT)])
    pltpu.sync_copy(fi_hbm.at[pl.ds(base, GPT)], idx_v)
    pltpu.async_copy(pe_hbm.at[idx_v], rows_v, sem).wait()
    pltpu.sync_copy(rows_v, outp.at[pl.ds(base, GPT)])


def _k2_gather(node_embed, h_idx_pad, pos_embed, f_i_pad):
    fn = pl.kernel(
        _k2_gather_body,
        out_type=(jax.ShapeDtypeStruct((NPAD, D), jnp.float32),
                  jax.ShapeDtypeStruct((NPAD, D), jnp.float32)),
        mesh=_mesh(),
        scratch_types=[
            pltpu.VMEM((GPT,), jnp.int32),
            pltpu.VMEM((GPT, D), jnp.float32),
            pltpu.SemaphoreType.DMA,
        ],
    )
    outh, outp = fn(node_embed, h_idx_pad, pos_embed, f_i_pad)
    return outh[:N], outp[:N]


def _k3_body(h_ref, p_ref, eemb_ref,
             a1w, a1b, b1w, b1b, b2w, b2b, c1w, c1b, c2w, c2b,
             a2wh, a2wp, a2b, b3w, b3b,
             a1o, b1o, b2o, c1o, c2o, a2o, b3to):
    h = h_ref[...]
    p = p_ref[...]
    a1o[...] = _mm(h, a1w, a1b)
    b1o[...] = _mm(h, b1w, b1b)
    b2o[...] = _mm(h, b2w, b2b)
    c1o[...] = _mm(p, c1w, c1b)
    c2o[...] = _mm(p, c2w, c2b)
    a2o[...] = _mm(h, a2wh, a2b) + lax.dot_general(
        p, a2wp[...], (((1,), (1,)), ((), ())),
        preferred_element_type=jnp.float32)

    @pl.when(pl.program_id(0) == 0)
    def _():
        b3to[...] = _mm(eemb_ref[...], b3w, b3b)


def _k3_node_mats(h0, p0, eemb_pad, lp):
    full = lambda shp: pl.BlockSpec(shp, lambda i: (0, 0))
    blk = pl.BlockSpec((1000, D), lambda i: (i, 0))
    r1 = lambda b: b.reshape(1, D)
    return pl.pallas_call(
        _k3_body,
        grid=(10,),
        in_specs=[blk, blk, full((8, D))] + [full(s) for s in
                  [(D, D), (1, D), (D, D), (1, D), (D, D), (1, D),
                   (D, D), (1, D), (D, D), (1, D),
                   (D, D), (D, D), (1, D), (D, D), (1, D)]],
        out_specs=[blk] * 6 + [pl.BlockSpec((8, D), lambda i: (0, 0))],
        out_shape=[jax.ShapeDtypeStruct((N, D), jnp.float32)] * 6
        + [jax.ShapeDtypeStruct((8, D), jnp.float32)],
    )(h0, p0, eemb_pad,
      lp["A1_w"], r1(lp["A1_b"]), lp["B1_w"], r1(lp["B1_b"]),
      lp["B2_w"], r1(lp["B2_b"]), lp["C1_w"], r1(lp["C1_b"]),
      lp["C2_w"], r1(lp["C2_b"]),
      lp["A2_w"][:, :D], lp["A2_w"][:, D:], r1(lp["A2_b"]),
      lp["B3_w"], r1(lp["B3_b"]))


def _pass1_body(with_lin, with_stats,
                b1h_hbm, b2h_hbm, b3t_hbm, lin_hbm, src_hbm, dst_hbm,
                eid_hbm, zer_hbm,
                sigout_hbm, ss0_hbm, ss1_hbm, stats_hbm,
                sidx, didx, eidx, b1, b2, b3, b4, sbuf,
                shared, s1, s2, s3, s4):
    cid = lax.axis_index("c")
    sid = lax.axis_index("s")
    wid = sid * NC + cid
    pltpu.sync_copy(zer_hbm, shared.at[pl.ds(sid * RPT, RPT)])
    plsc.subcore_barrier()

    base = wid * EPT_ALL
    nch = EPT_ALL // C
    zv = jnp.zeros((16,), jnp.float32)
    stats0 = tuple(zv for _ in range(16))

    def chunk(k, stats):
        off = base + k * C
        pltpu.sync_copy(src_hbm.at[pl.ds(off, C)], sidx)
        pltpu.sync_copy(dst_hbm.at[pl.ds(off, C)], didx)
        pltpu.sync_copy(eid_hbm.at[pl.ds(off, C)], eidx)
        d1 = pltpu.async_copy(b1h_hbm.at[sidx], b1, s1)
        d2 = pltpu.async_copy(b2h_hbm.at[didx], b2, s2)
        d3 = pltpu.async_copy(b3t_hbm.at[eidx], b3, s3)
        if with_lin:
            d4 = pltpu.async_copy(lin_hbm.at[pl.ds(off, C)], b4, s4)
        d1.wait()
        d2.wait()
        d3.wait()
        if with_lin:
            d4.wait()

        def row(r, st):
            sums = []
            sqs = []
            for kk in range(8):
                sl = pl.ds(kk * 16, 16)
                v = b1[r, sl] + b2[r, sl] + b3[r, sl]
                if with_lin:
                    v = v + b4[r, sl]
                s = _sigmoid(v)
                b1[r, sl] = s
                if with_stats:
                    b3[r, sl] = v
                    sums.append(st[kk] + v)
                    sqs.append(st[8 + kk] + v * v)
            if with_stats:
                return tuple(sums) + tuple(sqs)
            return st

        stats = lax.fori_loop(0, C, row, stats)
        if with_stats:
            pltpu.sync_copy(b3, sigout_hbm.at[pl.ds(off, C)])
        else:
            pltpu.sync_copy(b1, sigout_hbm.at[pl.ds(off, C)])
        pltpu.sync_copy(b1, shared.at[didx], add=True)
        return stats

    stats = lax.fori_loop(0, nch, chunk, stats0)
    if with_stats:
        for kk in range(8):
            sbuf[0, pl.ds(kk * 16, 16)] = stats[kk]
            sbuf[1, pl.ds(kk * 16, 16)] = stats[8 + kk]
            for rr in range(2, 8):
                sbuf[rr, pl.ds(kk * 16, 16)] = zv
        pltpu.sync_copy(sbuf, stats_hbm.at[pl.ds(wid * 8, 8)])

    plsc.subcore_barrier()
    rows = pl.ds(sid * RPT, RPT)

    @pl.when(cid == 0)
    def _():
        pltpu.sync_copy(shared.at[rows], ss0_hbm.at[rows])

    @pl.when(cid == 1)
    def _():
        pltpu.sync_copy(shared.at[rows], ss1_hbm.at[rows])


def _sc_pass1(b1h, b2h, b3tab, lin, src, dst, eid, zer, with_lin, with_stats):
    body = functools.partial(_pass1_body, with_lin, with_stats)
    fn = pl.kernel(
        body,
        out_type=(jax.ShapeDtypeStruct((E, D), jnp.float32),
                  jax.ShapeDtypeStruct((NP, D), jnp.float32),
                  jax.ShapeDtypeStruct((NP, D), jnp.float32),
                  jax.ShapeDtypeStruct((NW * 8, D), jnp.float32)),
        mesh=_mesh(),
        scratch_types=[
            pltpu.VMEM((C,), jnp.int32),
            pltpu.VMEM((C,), jnp.int32),
            pltpu.VMEM((C,), jnp.int32),
            pltpu.VMEM((C, D), jnp.float32),
            pltpu.VMEM((C, D), jnp.float32),
            pltpu.VMEM((C, D), jnp.float32),
            pltpu.VMEM((C, D), jnp.float32),
            pltpu.VMEM((8, D), jnp.float32),
            pltpu.VMEM_SHARED((NP, D), jnp.float32),
            pltpu.SemaphoreType.DMA,
            pltpu.SemaphoreType.DMA,
            pltpu.SemaphoreType.DMA,
            pltpu.SemaphoreType.DMA,
        ],
    )
    return fn(b1h, b2h, b3tab, lin, src, dst, eid, zer)


def _recip_body(a_ref, b_ref, o_ref):
    o_ref[...] = 1.0 / (a_ref[...] + b_ref[...] + 1e-6)


def _k5_recip(ss0, ss1):
    blk = pl.BlockSpec((1024, D), lambda i: (i, 0))
    return pl.pallas_call(
        _recip_body, grid=(10,), in_specs=[blk, blk], out_specs=blk,
        out_shape=jax.ShapeDtypeStruct((NP, D), jnp.float32),
    )(ss0, ss1)


def _k6_body(hat_hbm, rec_hbm, a2v_hbm, c2p_hbm, src_hbm, dst_hbm, zer_hbm,
             hagg_hbm, pagg_hbm,
             sidx, didx, hb, rb, vb, shared, s1, s2, s3):
    cid = lax.axis_index("c")
    sid = lax.axis_index("s")
    pltpu.sync_copy(zer_hbm, shared.at[pl.ds(sid * RPT, RPT)])
    plsc.subcore_barrier()

    base = sid * EPT_CORE
    nch = EPT_CORE // C

    def chunk(k, carry):
        off = base + k * C
        pltpu.sync_copy(src_hbm.at[pl.ds(off, C)], sidx)
        pltpu.sync_copy(dst_hbm.at[pl.ds(off, C)], didx)
        d1 = pltpu.async_copy(hat_hbm.at[pl.ds(off, C)], hb, s1)
        d2 = pltpu.async_copy(rec_hbm.at[didx], rb, s2)

        @pl.when(cid == 0)
        def _():
            pltpu.async_copy(a2v_hbm.at[sidx], vb, s3).wait()

        @pl.when(cid == 1)
        def _():
            pltpu.async_copy(c2p_hbm.at[sidx], vb, s3).wait()
        d1.wait()
        d2.wait()

        def row(r, c):
            for kk in range(8):
                sl = pl.ds(kk * 16, 16)
                eta = _sigmoid(hb[r, sl]) * rb[r, sl]
                vb[r, sl] = eta * vb[r, sl]
            return c

        lax.fori_loop(0, C, row, 0)
        pltpu.sync_copy(vb, shared.at[didx], add=True)
        return carry

    lax.fori_loop(0, nch, chunk, 0)
    plsc.subcore_barrier()
    rows = pl.ds(sid * RPT, RPT)

    @pl.when(cid == 0)
    def _():
        pltpu.sync_copy(shared.at[rows], hagg_hbm.at[rows])

    @pl.when(cid == 1)
    def _():
        pltpu.sync_copy(shared.at[rows], pagg_hbm.at[rows])


def _k6_pass2_l1(hat1, recip1, a2v, c2p, src, dst, zer):
    fn = pl.kernel(
        _k6_body,
        out_type=(jax.ShapeDtypeStruct((NP, D), jnp.float32),
                  jax.ShapeDtypeStruct((NP, D), jnp.float32)),
        mesh=_mesh(),
        scratch_types=[
            pltpu.VMEM((C,), jnp.int32),
            pltpu.VMEM((C,), jnp.int32),
            pltpu.VMEM((C, D), jnp.float32),
            pltpu.VMEM((C, D), jnp.float32),
            pltpu.VMEM((C, D), jnp.float32),
            pltpu.VMEM_SHARED((NP, D), jnp.float32),
            pltpu.SemaphoreType.DMA,
            pltpu.SemaphoreType.DMA,
            pltpu.SemaphoreType.DMA,
        ],
    )
    return fn(hat1, recip1, a2v, c2p, src, dst, zer)


def _k11_body(sig_hbm, rec_hbm, a2v_hbm, src_hbm, dst_hbm, zer_hbm,
              h0_hbm, h1_hbm,
              sidx, didx, sb, rb, vb, shared, s1, s2, s3):
    cid = lax.axis_index("c")
    sid = lax.axis_index("s")
    wid = sid * NC + cid
    pltpu.sync_copy(zer_hbm, shared.at[pl.ds(sid * RPT, RPT)])
    plsc.subcore_barrier()

    base = wid * EPT_ALL
    nch = EPT_ALL // C

    def chunk(k, carry):
        off = base + k * C
        pltpu.sync_copy(src_hbm.at[pl.ds(off, C)], sidx)
        pltpu.sync_copy(dst_hbm.at[pl.ds(off, C)], didx)
        d1 = pltpu.async_copy(sig_hbm.at[pl.ds(off, C)], sb, s1)
        d2 = pltpu.async_copy(rec_hbm.at[didx], rb, s2)
        d3 = pltpu.async_copy(a2v_hbm.at[sidx], vb, s3)
        d1.wait()
        d2.wait()
        d3.wait()

        def row(r, c):
            for kk in range(8):
                sl = pl.ds(kk * 16, 16)
                vb[r, sl] = sb[r, sl] * rb[r, sl] * vb[r, sl]
            return c

        lax.fori_loop(0, C, row, 0)
        pltpu.sync_copy(vb, shared.at[didx], add=True)
        return carry

    lax.fori_loop(0, nch, chunk, 0)
    plsc.subcore_barrier()
    rows = pl.ds(sid * RPT, RPT)

    @pl.when(cid == 0)
    def _():
        pltpu.sync_copy(shared.at[rows], h0_hbm.at[rows])

    @pl.when(cid == 1)
    def _():
        pltpu.sync_copy(shared.at[rows], h1_hbm.at[rows])


def _k11_pass2_l2(sig2, recip2, a2v2, src, dst, zer):
    fn = pl.kernel(
        _k11_body,
        out_type=(jax.ShapeDtypeStruct((NP, D), jnp.float32),
                  jax.ShapeDtypeStruct((NP, D), jnp.float32)),
        mesh=_mesh(),
        scratch_types=[
            pltpu.VMEM((C,), jnp.int32),
            pltpu.VMEM((C,), jnp.int32),
            pltpu.VMEM((C, D), jnp.float32),
            pltpu.VMEM((C, D), jnp.float32),
            pltpu.VMEM((C, D), jnp.float32),
            pltpu.VMEM_SHARED((NP, D), jnp.float32),
            pltpu.SemaphoreType.DMA,
            pltpu.SemaphoreType.DMA,
            pltpu.SemaphoreType.DMA,
        ],
    )
    return fn(sig2, recip2, a2v2, src, dst, zer)


def _stats_body(nparts, *refs):
    in_refs = refs[:nparts]
    o_ref = refs[nparts]
    acc = refs[nparts + 1]
    i = pl.program_id(0)

    @pl.when(i == 0)
    def _():
        acc[...] = jnp.zeros_like(acc)

    x = in_refs[0][...]
    for r in in_refs[1:]:
        x = x + r[...]
    acc[0:1, :] += jnp.sum(x, axis=0, keepdims=True)
    acc[1:2, :] += jnp.sum(x * x, axis=0, keepdims=True)

    @pl.when(i == pl.num_programs(0) - 1)
    def _():
        o_ref[...] = acc[...]


def _k_stats(*parts):
    blk = pl.BlockSpec((1000, D), lambda i: (i, 0))
    body = functools.partial(_stats_body, len(parts))
    return pl.pallas_call(
        body, grid=(10,),
        in_specs=[blk] * len(parts),
        out_specs=pl.BlockSpec((8, D), lambda i: (0, 0)),
        out_shape=jax.ShapeDtypeStruct((8, D), jnp.float32),
        scratch_shapes=[pltpu.VMEM((8, D), jnp.float32)],
    )(*parts)


def _k7b_body(a1_ref, hagg_ref, h0_ref, c1_ref, pagg_ref, p0_ref, st_ref,
              g_ref, bb_ref,
              a1w, a1b, b1w, b1b, b2w, b2b, a2wh, a2wp, a2b,
              h1o, a1o, b1o, b2o, a2o):
    xh = a1_ref[...] + hagg_ref[...]
    mu = st_ref[0:1, :] * (1.0 / N)
    var = st_ref[1:2, :] * (1.0 / N) - mu * mu
    hn = g_ref[...] * (xh - mu) * lax.rsqrt(var + 1e-5) + bb_ref[...]
    h1 = h0_ref[...] + jnp.maximum(hn, 0.0)
    p1 = p0_ref[...] + jnp.tanh(c1_ref[...] + pagg_ref[...])
    h1o[...] = h1
    a1o[...] = _mm(h1, a1w, a1b)
    b1o[...] = _mm(h1, b1w, b1b)
    b2o[...] = _mm(h1, b2w, b2b)
    a2o[...] = _mm(h1, a2wh, a2b) + lax.dot_general(
        p1, a2wp[...], (((1,), (1,)), ((), ())),
        preferred_element_type=jnp.float32)


def _k7b(a1h1, hagg, h0, c1p1, pagg, p0, stats, lp1, lp2):
    blk = pl.BlockSpec((1000, D), lambda i: (i, 0))
    full = lambda shp: pl.BlockSpec(shp, lambda i: (0, 0))
    r1 = lambda b: b.reshape(1, D)
    return pl.pallas_call(
        _k7b_body,
        grid=(10,),
        in_specs=[blk, blk, blk, blk, blk, blk, full((8, D)),
                  full((1, D)), full((1, D))] + [full(s) for s in
                  [(D, D), (1, D), (D, D), (1, D), (D, D), (1, D),
                   (D, D), (D, D), (1, D)]],
        out_specs=[blk] * 5,
        out_shape=[jax.ShapeDtypeStruct((N, D), jnp.float32)] * 5,
    )(a1h1, hagg, h0, c1p1, pagg, p0, stats,
      lp1["bn_h_g"].reshape(1, D), lp1["bn_h_b"].reshape(1, D),
      lp2["A1_w"], r1(lp2["A1_b"]), lp2["B1_w"], r1(lp2["B1_b"]),
      lp2["B2_w"], r1(lp2["B2_b"]),
      lp2["A2_w"][:, :D], lp2["A2_w"][:, D:], r1(lp2["A2_b"]))


def _k8_body(hat_ref, st_ref, eemb_ref, g_ref, bb_ref, b3w, b3b,
             out_ref, eb3_ref):
    st = st_ref[...]
    tot = jnp.sum(st.reshape(NW, 8, D), axis=0)
    mu = tot[0:1, :] * (1.0 / E)
    var = tot[1:2, :] * (1.0 / E) - mu * mu
    x = g_ref[...] * (hat_ref[...] - mu) * lax.rsqrt(var + 1e-5) + bb_ref[...]
    x = jnp.maximum(x, 0.0)
    out_ref[...] = _mm(x, b3w, b3b)

    @pl.when(pl.program_id(0) == 0)
    def _():
        eb3_ref[...] = lax.dot_general(
            eemb_ref[...], b3w[...], (((1,), (1,)), ((), ())),
            preferred_element_type=jnp.float32)


def _k8(hat1, stats_e, eemb_pad, lp1, lp2):
    blk = pl.BlockSpec((1000, D), lambda i: (i, 0))
    full = lambda shp: pl.BlockSpec(shp, lambda i: (0, 0))
    return pl.pallas_call(
        _k8_body,
        grid=(160,),
        in_specs=[blk, full((8 * NW, D)), full((8, D)), full((1, D)),
                  full((1, D)), full((D, D)), full((1, D))],
        out_specs=[blk, pl.BlockSpec((8, D), lambda i: (0, 0))],
        out_shape=[jax.ShapeDtypeStruct((E, D), jnp.float32),
                   jax.ShapeDtypeStruct((8, D), jnp.float32)],
    )(hat1, stats_e, eemb_pad,
      lp1["bn_e_g"].reshape(1, D), lp1["bn_e_b"].reshape(1, D),
      lp2["B3_w"], lp2["B3_b"].reshape(1, D))


def _k12b_body(a1_ref, g0_ref, g1_ref, h1_ref, st_ref, g_ref, bb_ref,
               w1, b1, w2, b2, w3, b3,
               y_ref, acc):
    i = pl.program_id(0)

    @pl.when(i == 0)
    def _():
        acc[...] = jnp.zeros_like(acc)

    xh = a1_ref[...] + g0_ref[...] + g1_ref[...]
    mu = st_ref[0:1, :] * (1.0 / N)
    var = st_ref[1:2, :] * (1.0 / N) - mu * mu
    hn = g_ref[...] * (xh - mu) * lax.rsqrt(var + 1e-5) + bb_ref[...]
    h2 = h1_ref[...] + jnp.maximum(hn, 0.0)
    acc[0:1, :] += jnp.sum(h2, axis=0, keepdims=True)

    @pl.when(i == pl.num_programs(0) - 1)
    def _():
        hg = acc[0:1, :] * (1.0 / N)
        y1 = jnp.maximum(_mm(hg, w1, b1), 0.0)
        y2 = jnp.maximum(_mm(y1, w2, b2), 0.0)
        y3 = lax.dot_general(y2, w3[...], (((1,), (1,)), ((), ())),
                             preferred_element_type=jnp.float32)
        y_ref[...] = y3[0:1, 0:1] + b3[0:1, 0:1]


def _k12b(a1h2, hg0, hg1, h1, stats, lp2, mlp):
    blk = pl.BlockSpec((1000, D), lambda i: (i, 0))
    full = lambda shp: pl.BlockSpec(shp, lambda i: (0, 0))
    (w1, b1), (w2, b2), (w3, b3) = mlp
    w1f = jnp.zeros((D, D), jnp.float32).at[:64, :].set(w1)
    b1p = jnp.zeros((1, D), jnp.float32).at[0, :64].set(b1)
    w2f = jnp.zeros((D, D), jnp.float32).at[:32, :64].set(w2)
    b2p = jnp.zeros((1, D), jnp.float32).at[0, :32].set(b2)
    w3f = jnp.zeros((8, D), jnp.float32).at[0:1, :32].set(w3)
    b3p = jnp.zeros((1, D), jnp.float32).at[0, 0].set(b3[0])
    return pl.pallas_call(
        _k12b_body,
        grid=(10,),
        in_specs=[blk, blk, blk, blk, full((8, D)), full((1, D)),
                  full((1, D)), full((D, D)), full((1, D)), full((D, D)),
                  full((1, D)), full((8, D)), full((1, D))],
        out_specs=pl.BlockSpec((1, 1), lambda i: (0, 0)),
        out_shape=jax.ShapeDtypeStruct((1, 1), jnp.float32),
        scratch_shapes=[pltpu.VMEM((8, D), jnp.float32)],
    )(a1h2, hg0, hg1, h1, stats,
      lp2["bn_h_g"].reshape(1, D), lp2["bn_h_b"].reshape(1, D),
      w1f, b1p, w2f, b2p, w3f, b3p)


def kernel(f, params, h, e, edge_index):
    lp1, lp2 = params["layers"]
    src = edge_index[0]
    dst = edge_index[1]
    eid = e

    f_i = _k1_findex(params["Wf"], f, params["bf"])

    pad = jnp.zeros((NPAD - N,), jnp.int32)
    h0, p0 = _k2_gather(params["node_embed"],
                        jnp.concatenate([h, pad]),
                        params["pos_embed"],
                        jnp.concatenate([f_i, pad]))

    eemb_pad = jnp.zeros((8, D), jnp.float32).at[:NBOND].set(
        params["edge_embed"])
    zer = jnp.zeros((RPT, D), jnp.float32)

    a1h1, b1h1, b2h1, c1p1, c2p1, a2v1, b3e1 = _k3_node_mats(
        h0, p0, eemb_pad, lp1)

    hat1, ss0, ss1, stats_e = _sc_pass1(
        b1h1, b2h1, b3e1, b3e1, src, dst, eid, zer,
        with_lin=False, with_stats=True)

    recip1 = _k5_recip(ss0, ss1)

    hagg1, pagg1 = _k6_pass2_l1(hat1, recip1, a2v1, c2p1, src, dst, zer)
    hagg1 = hagg1[:N]
    pagg1 = pagg1[:N]

    stats_h1 = _k_stats(a1h1, hagg1)
    h1, a1h2, b1h2, b2h2, a2v2 = _k7b(
        a1h1, hagg1, h0, c1p1, pagg1, p0, stats_h1, lp1, lp2)

    b3r, eb3 = _k8(hat1, stats_e, eemb_pad, lp1, lp2)

    sig2, ss0b, ss1b, _unused_stats = _sc_pass1(
        b1h2, b2h2, eb3, b3r, src, dst, eid, zer,
        with_lin=True, with_stats=False)

    recip2 = _k5_recip(ss0b, ss1b)

    hg2a, hg2b = _k11_pass2_l2(sig2, recip2, a2v2, src, dst, zer)
    hg2a = hg2a[:N]
    hg2b = hg2b[:N]

    stats_h2 = _k_stats(a1h2, hg2a, hg2b)
    y = _k12b(a1h2, hg2a, hg2b, h1, stats_h2, lp2, params["mlp"])
    return y


# speedup vs baseline: 1.9301x; 1.9301x over previous
import functools

import jax
import jax.numpy as jnp
from jax import lax
from jax.experimental import pallas as pl
from jax.experimental.pallas import tpu as pltpu
from jax.experimental.pallas import tpu_sc as plsc

N = 10000
E = 160000
D = 128
NBOND = 4

NC = 2
NS = 16
NW = NC * NS

C = 40
EPT_ALL = E // NW
EPT_CORE = E // NS
NP = 10240
RPT = NP // NS
GPT = 320
NPAD = GPT * NW

_mesh = functools.partial(
    plsc.VectorSubcoreMesh, core_axis_name="c", subcore_axis_name="s",
    num_cores=NC, num_subcores=NS)


def _sigmoid(x):
    return 1.0 / (1.0 + jnp.exp(-x))


def _mm(x, w_ref, b_ref):
    return lax.dot_general(x, w_ref[...], (((1,), (1,)), ((), ())),
                           preferred_element_type=jnp.float32) + b_ref[...]


def _matvec_body(wf_ref, f_ref, bf_ref, o_ref):
    r = lax.dot_general(
        wf_ref[...], f_ref[...], (((1,), (1,)), ((), ())),
        preferred_element_type=jnp.float32)
    v = r[:, 0:1] + bf_ref[:, 0:1]
    iv = jnp.clip(jnp.abs(v.astype(jnp.int32)), 0, N - 1)
    o_ref[...] = jnp.broadcast_to(iv, (400, 128))


def _k1_findex(Wf, f, bf):
    f2 = jnp.broadcast_to(f[None, :], (8, N))
    bf2 = jnp.broadcast_to(bf[:, None], (N, 128))
    out = pl.pallas_call(
        _matvec_body,
        grid=(25,),
        in_specs=[
            pl.BlockSpec((400, N), lambda i: (i, 0)),
            pl.BlockSpec((8, N), lambda i: (0, 0)),
            pl.BlockSpec((400, 128), lambda i: (i, 0)),
        ],
        out_specs=pl.BlockSpec((400, 128), lambda i: (i, 0)),
        out_shape=jax.ShapeDtypeStruct((N, 128), jnp.int32),
    )(Wf, f2, bf2)
    return out[:, 0]


def _k2_gather_body(ne_hbm, hidx_hbm, pe_hbm, fi_hbm, outh, outp,
                    idx_v, rows_v, sem):
    wid = lax.axis_index("s") * NC + lax.axis_index("c")
    base = wid * GPT
    pltpu.sync_copy(hidx_hbm.at[pl.ds(base, GPT)], idx_v)
    pltpu.async_copy(ne_hbm.at[idx_v], rows_v, sem).wait()
    pltpu.sync_copy(rows_v, outh.at[pl.ds(base, GPT)])
    pltpu.sync_copy(fi_hbm.at[pl.ds(base, GPT)], idx_v)
    pltpu.async_copy(pe_hbm.at[idx_v], rows_v, sem).wait()
    pltpu.sync_copy(rows_v, outp.at[pl.ds(base, GPT)])


def _k2_gather(node_embed, h_idx_pad, pos_embed, f_i_pad):
    fn = pl.kernel(
        _k2_gather_body,
        out_type=(jax.ShapeDtypeStruct((NPAD, D), jnp.float32),
                  jax.ShapeDtypeStruct((NPAD, D), jnp.float32)),
        mesh=_mesh(),
        scratch_types=[
            pltpu.VMEM((GPT,), jnp.int32),
            pltpu.VMEM((GPT, D), jnp.float32),
            pltpu.SemaphoreType.DMA,
        ],
    )
    outh, outp = fn(node_embed, h_idx_pad, pos_embed, f_i_pad)
    return outh[:N], outp[:N]


def _k3_body(h_ref, p_ref, eemb_ref,
             a1w, a1b, b1w, b1b, b2w, b2b, c1w, c1b, c2w, c2b,
             a2wh, a2wp, a2b, b3w, b3b,
             a1o, b1o, b2o, c1o, c2o, a2o, b3to):
    h = h_ref[...]
    p = p_ref[...]
    a1o[...] = _mm(h, a1w, a1b)
    b1o[...] = _mm(h, b1w, b1b)
    b2o[...] = _mm(h, b2w, b2b)
    c1o[...] = _mm(p, c1w, c1b)
    c2o[...] = _mm(p, c2w, c2b)
    a2o[...] = _mm(h, a2wh, a2b) + lax.dot_general(
        p, a2wp[...], (((1,), (1,)), ((), ())),
        preferred_element_type=jnp.float32)

    @pl.when(pl.program_id(0) == 0)
    def _():
        b3to[...] = _mm(eemb_ref[...], b3w, b3b)


def _k3_node_mats(h0, p0, eemb_pad, lp):
    full = lambda shp: pl.BlockSpec(shp, lambda i: (0, 0))
    blk = pl.BlockSpec((1000, D), lambda i: (i, 0))
    r1 = lambda b: b.reshape(1, D)
    return pl.pallas_call(
        _k3_body,
        grid=(10,),
        in_specs=[blk, blk, full((8, D))] + [full(s) for s in
                  [(D, D), (1, D), (D, D), (1, D), (D, D), (1, D),
                   (D, D), (1, D), (D, D), (1, D),
                   (D, D), (D, D), (1, D), (D, D), (1, D)]],
        out_specs=[blk] * 6 + [pl.BlockSpec((8, D), lambda i: (0, 0))],
        out_shape=[jax.ShapeDtypeStruct((N, D), jnp.float32)] * 6
        + [jax.ShapeDtypeStruct((8, D), jnp.float32)],
    )(h0, p0, eemb_pad,
      lp["A1_w"], r1(lp["A1_b"]), lp["B1_w"], r1(lp["B1_b"]),
      lp["B2_w"], r1(lp["B2_b"]), lp["C1_w"], r1(lp["C1_b"]),
      lp["C2_w"], r1(lp["C2_b"]),
      lp["A2_w"][:, :D], lp["A2_w"][:, D:], r1(lp["A2_b"]),
      lp["B3_w"], r1(lp["B3_b"]))


def _pass1_body(with_lin, with_stats,
                b1h_hbm, b2h_hbm, b3t_hbm, lin_hbm, src_hbm, dst_hbm,
                eid_hbm, zer_hbm,
                sigout_hbm, ss0_hbm, ss1_hbm, stats_hbm,
                sidx, didx, eidx, b1, b2, b3, b4, sbuf,
                shared, s1, s2, s3, s4):
    cid = lax.axis_index("c")
    sid = lax.axis_index("s")
    wid = sid * NC + cid
    pltpu.sync_copy(zer_hbm, shared.at[pl.ds(sid * RPT, RPT)])
    plsc.subcore_barrier()

    base = wid * EPT_ALL
    nch = EPT_ALL // C
    zv = jnp.zeros((16,), jnp.float32)
    stats0 = tuple(zv for _ in range(16))

    def chunk(k, stats):
        off = base + k * C
        pltpu.sync_copy(src_hbm.at[pl.ds(off, C)], sidx)
        pltpu.sync_copy(dst_hbm.at[pl.ds(off, C)], didx)
        pltpu.sync_copy(eid_hbm.at[pl.ds(off, C)], eidx)
        d1 = pltpu.async_copy(b1h_hbm.at[sidx], b1, s1)
        d2 = pltpu.async_copy(b2h_hbm.at[didx], b2, s2)
        d3 = pltpu.async_copy(b3t_hbm.at[eidx], b3, s3)
        if with_lin:
            d4 = pltpu.async_copy(lin_hbm.at[pl.ds(off, C)], b4, s4)
        d1.wait()
        d2.wait()
        d3.wait()
        if with_lin:
            d4.wait()

        def row(r, st):
            sums = []
            sqs = []
            for kk in range(8):
                sl = pl.ds(kk * 16, 16)
                v = b1[r, sl] + b2[r, sl] + b3[r, sl]
                if with_lin:
                    v = v + b4[r, sl]
                s = _sigmoid(v)
                b1[r, sl] = s
                if with_stats:
                    b3[r, sl] = v
                    sums.append(st[kk] + v)
                    sqs.append(st[8 + kk] + v * v)
            if with_stats:
                return tuple(sums) + tuple(sqs)
            return st

        stats = lax.fori_loop(0, C, row, stats)
        if with_stats:
            pltpu.sync_copy(b3, sigout_hbm.at[pl.ds(off, C)])
        else:
            pltpu.sync_copy(b1, sigout_hbm.at[pl.ds(off, C)])
        pltpu.sync_copy(b1, shared.at[didx], add=True)
        return stats

    stats = lax.fori_loop(0, nch, chunk, stats0)
    if with_stats:
        for kk in range(8):
            sbuf[0, pl.ds(kk * 16, 16)] = stats[kk]
            sbuf[1, pl.ds(kk * 16, 16)] = stats[8 + kk]
            for rr in range(2, 8):
                sbuf[rr, pl.ds(kk * 16, 16)] = zv
        pltpu.sync_copy(sbuf, stats_hbm.at[pl.ds(wid * 8, 8)])

    plsc.subcore_barrier()
    rows = pl.ds(sid * RPT, RPT)

    @pl.when(cid == 0)
    def _():
        pltpu.sync_copy(shared.at[rows], ss0_hbm.at[rows])

    @pl.when(cid == 1)
    def _():
        pltpu.sync_copy(shared.at[rows], ss1_hbm.at[rows])


def _sc_pass1(b1h, b2h, b3tab, lin, src, dst, eid, zer, with_lin, with_stats):
    body = functools.partial(_pass1_body, with_lin, with_stats)
    fn = pl.kernel(
        body,
        out_type=(jax.ShapeDtypeStruct((E, D), jnp.float32),
                  jax.ShapeDtypeStruct((NP, D), jnp.float32),
                  jax.ShapeDtypeStruct((NP, D), jnp.float32),
                  jax.ShapeDtypeStruct((NW * 8, D), jnp.float32)),
        mesh=_mesh(),
        scratch_types=[
            pltpu.VMEM((C,), jnp.int32),
            pltpu.VMEM((C,), jnp.int32),
            pltpu.VMEM((C,), jnp.int32),
            pltpu.VMEM((C, D), jnp.float32),
            pltpu.VMEM((C, D), jnp.float32),
            pltpu.VMEM((C, D), jnp.float32),
            pltpu.VMEM((C, D), jnp.float32),
            pltpu.VMEM((8, D), jnp.float32),
            pltpu.VMEM_SHARED((NP, D), jnp.float32),
            pltpu.SemaphoreType.DMA,
            pltpu.SemaphoreType.DMA,
            pltpu.SemaphoreType.DMA,
            pltpu.SemaphoreType.DMA,
        ],
    )
    return fn(b1h, b2h, b3tab, lin, src, dst, eid, zer)


def _recip_body(a_ref, b_ref, o_ref):
    o_ref[...] = 1.0 / (a_ref[...] + b_ref[...] + 1e-6)


def _k5_recip(ss0, ss1):
    blk = pl.BlockSpec((1024, D), lambda i: (i, 0))
    return pl.pallas_call(
        _recip_body, grid=(10,), in_specs=[blk, blk], out_specs=blk,
        out_shape=jax.ShapeDtypeStruct((NP, D), jnp.float32),
    )(ss0, ss1)


def _k6_body(hat_hbm, rec_hbm, a2v_hbm, c2p_hbm, src_hbm, dst_hbm, zer_hbm,
             hagg_hbm, pagg_hbm,
             sidx, didx, hb, rb, vb, shared, s1, s2, s3):
    cid = lax.axis_index("c")
    sid = lax.axis_index("s")
    pltpu.sync_copy(zer_hbm, shared.at[pl.ds(sid * RPT, RPT)])
    plsc.subcore_barrier()

    base = sid * EPT_CORE
    nch = EPT_CORE // C

    def chunk(k, carry):
        off = base + k * C
        pltpu.sync_copy(src_hbm.at[pl.ds(off, C)], sidx)
        pltpu.sync_copy(dst_hbm.at[pl.ds(off, C)], didx)
        d1 = pltpu.async_copy(hat_hbm.at[pl.ds(off, C)], hb, s1)
        d2 = pltpu.async_copy(rec_hbm.at[didx], rb, s2)

        @pl.when(cid == 0)
        def _():
            pltpu.async_copy(a2v_hbm.at[sidx], vb, s3).wait()

        @pl.when(cid == 1)
        def _():
            pltpu.async_copy(c2p_hbm.at[sidx], vb, s3).wait()
        d1.wait()
        d2.wait()

        def row(r, c):
            for kk in range(8):
                sl = pl.ds(kk * 16, 16)
                eta = _sigmoid(hb[r, sl]) * rb[r, sl]
                vb[r, sl] = eta * vb[r, sl]
            return c

        lax.fori_loop(0, C, row, 0)
        pltpu.sync_copy(vb, shared.at[didx], add=True)
        return carry

    lax.fori_loop(0, nch, chunk, 0)
    plsc.subcore_barrier()
    rows = pl.ds(sid * RPT, RPT)

    @pl.when(cid == 0)
    def _():
        pltpu.sync_copy(shared.at[rows], hagg_hbm.at[rows])

    @pl.when(cid == 1)
    def _():
        pltpu.sync_copy(shared.at[rows], pagg_hbm.at[rows])


def _k6_pass2_l1(hat1, recip1, a2v, c2p, src, dst, zer):
    fn = pl.kernel(
        _k6_body,
        out_type=(jax.ShapeDtypeStruct((NP, D), jnp.float32),
                  jax.ShapeDtypeStruct((NP, D), jnp.float32)),
        mesh=_mesh(),
        scratch_types=[
            pltpu.VMEM((C,), jnp.int32),
            pltpu.VMEM((C,), jnp.int32),
            pltpu.VMEM((C, D), jnp.float32),
            pltpu.VMEM((C, D), jnp.float32),
            pltpu.VMEM((C, D), jnp.float32),
            pltpu.VMEM_SHARED((NP, D), jnp.float32),
            pltpu.SemaphoreType.DMA,
            pltpu.SemaphoreType.DMA,
            pltpu.SemaphoreType.DMA,
        ],
    )
    return fn(hat1, recip1, a2v, c2p, src, dst, zer)


def _k11_body(sig_hbm, rec_hbm, a2v_hbm, src_hbm, dst_hbm, zer_hbm,
              h0_hbm, h1_hbm,
              sidx, didx, sb, rb, vb, shared, s1, s2, s3):
    cid = lax.axis_index("c")
    sid = lax.axis_index("s")
    wid = sid * NC + cid
    pltpu.sync_copy(zer_hbm, shared.at[pl.ds(sid * RPT, RPT)])
    plsc.subcore_barrier()

    base = wid * EPT_ALL
    nch = EPT_ALL // C

    def chunk(k, carry):
        off = base + k * C
        pltpu.sync_copy(src_hbm.at[pl.ds(off, C)], sidx)
        pltpu.sync_copy(dst_hbm.at[pl.ds(off, C)], didx)
        d1 = pltpu.async_copy(sig_hbm.at[pl.ds(off, C)], sb, s1)
        d2 = pltpu.async_copy(rec_hbm.at[didx], rb, s2)
        d3 = pltpu.async_copy(a2v_hbm.at[sidx], vb, s3)
        d1.wait()
        d2.wait()
        d3.wait()

        def row(r, c):
            for kk in range(8):
                sl = pl.ds(kk * 16, 16)
                vb[r, sl] = sb[r, sl] * rb[r, sl] * vb[r, sl]
            return c

        lax.fori_loop(0, C, row, 0)
        pltpu.sync_copy(vb, shared.at[didx], add=True)
        return carry

    lax.fori_loop(0, nch, chunk, 0)
    plsc.subcore_barrier()
    rows = pl.ds(sid * RPT, RPT)

    @pl.when(cid == 0)
    def _():
        pltpu.sync_copy(shared.at[rows], h0_hbm.at[rows])

    @pl.when(cid == 1)
    def _():
        pltpu.sync_copy(shared.at[rows], h1_hbm.at[rows])


def _k11_pass2_l2(sig2, recip2, a2v2, src, dst, zer):
    fn = pl.kernel(
        _k11_body,
        out_type=(jax.ShapeDtypeStruct((NP, D), jnp.float32),
                  jax.ShapeDtypeStruct((NP, D), jnp.float32)),
        mesh=_mesh(),
        scratch_types=[
            pltpu.VMEM((C,), jnp.int32),
            pltpu.VMEM((C,), jnp.int32),
            pltpu.VMEM((C, D), jnp.float32),
            pltpu.VMEM((C, D), jnp.float32),
            pltpu.VMEM((C, D), jnp.float32),
            pltpu.VMEM_SHARED((NP, D), jnp.float32),
            pltpu.SemaphoreType.DMA,
            pltpu.SemaphoreType.DMA,
            pltpu.SemaphoreType.DMA,
        ],
    )
    return fn(sig2, recip2, a2v2, src, dst, zer)


def _stats_body(nparts, *refs):
    in_refs = refs[:nparts]
    o_ref = refs[nparts]
    acc = refs[nparts + 1]
    i = pl.program_id(0)

    @pl.when(i == 0)
    def _():
        acc[...] = jnp.zeros_like(acc)

    x = in_refs[0][...]
    for r in in_refs[1:]:
        x = x + r[...]
    acc[0:1, :] += jnp.sum(x, axis=0, keepdims=True)
    acc[1:2, :] += jnp.sum(x * x, axis=0, keepdims=True)

    @pl.when(i == pl.num_programs(0) - 1)
    def _():
        o_ref[...] = acc[...]


def _k_stats(*parts):
    blk = pl.BlockSpec((1000, D), lambda i: (i, 0))
    body = functools.partial(_stats_body, len(parts))
    return pl.pallas_call(
        body, grid=(10,),
        in_specs=[blk] * len(parts),
        out_specs=pl.BlockSpec((8, D), lambda i: (0, 0)),
        out_shape=jax.ShapeDtypeStruct((8, D), jnp.float32),
        scratch_shapes=[pltpu.VMEM((8, D), jnp.float32)],
    )(*parts)


def _k7b_body(a1_ref, hagg_ref, h0_ref, c1_ref, pagg_ref, p0_ref, st_ref,
              g_ref, bb_ref,
              a1w, a1b, b1w, b1b, b2w, b2b, a2wh, a2wp, a2b,
              h1o, a1o, b1o, b2o, a2o):
    xh = a1_ref[...] + hagg_ref[...]
    mu = st_ref[0:1, :] * (1.0 / N)
    var = st_ref[1:2, :] * (1.0 / N) - mu * mu
    hn = g_ref[...] * (xh - mu) * lax.rsqrt(var + 1e-5) + bb_ref[...]
    h1 = h0_ref[...] + jnp.maximum(hn, 0.0)
    p1 = p0_ref[...] + jnp.tanh(c1_ref[...] + pagg_ref[...])
    h1o[...] = h1
    a1o[...] = _mm(h1, a1w, a1b)
    b1o[...] = _mm(h1, b1w, b1b)
    b2o[...] = _mm(h1, b2w, b2b)
    a2o[...] = _mm(h1, a2wh, a2b) + lax.dot_general(
        p1, a2wp[...], (((1,), (1,)), ((), ())),
        preferred_element_type=jnp.float32)


def _k7b(a1h1, hagg, h0, c1p1, pagg, p0, stats, lp1, lp2):
    blk = pl.BlockSpec((1000, D), lambda i: (i, 0))
    full = lambda shp: pl.BlockSpec(shp, lambda i: (0, 0))
    r1 = lambda b: b.reshape(1, D)
    return pl.pallas_call(
        _k7b_body,
        grid=(10,),
        in_specs=[blk, blk, blk, blk, blk, blk, full((8, D)),
                  full((1, D)), full((1, D))] + [full(s) for s in
                  [(D, D), (1, D), (D, D), (1, D), (D, D), (1, D),
                   (D, D), (D, D), (1, D)]],
        out_specs=[blk] * 5,
        out_shape=[jax.ShapeDtypeStruct((N, D), jnp.float32)] * 5,
    )(a1h1, hagg, h0, c1p1, pagg, p0, stats,
      lp1["bn_h_g"].reshape(1, D), lp1["bn_h_b"].reshape(1, D),
      lp2["A1_w"], r1(lp2["A1_b"]), lp2["B1_w"], r1(lp2["B1_b"]),
      lp2["B2_w"], r1(lp2["B2_b"]),
      lp2["A2_w"][:, :D], lp2["A2_w"][:, D:], r1(lp2["A2_b"]))


def _k8_body(hat_ref, st_ref, eemb_ref, g_ref, bb_ref, b3w, b3b,
             out_ref, eb3_ref):
    st = st_ref[...]
    tot = jnp.sum(st.reshape(NW, 8, D), axis=0)
    mu = tot[0:1, :] * (1.0 / E)
    var = tot[1:2, :] * (1.0 / E) - mu * mu
    x = g_ref[...] * (hat_ref[...] - mu) * lax.rsqrt(var + 1e-5) + bb_ref[...]
    x = jnp.maximum(x, 0.0)
    out_ref[...] = _mm(x, b3w, b3b)

    @pl.when(pl.program_id(0) == 0)
    def _():
        eb3_ref[...] = lax.dot_general(
            eemb_ref[...], b3w[...], (((1,), (1,)), ((), ())),
            preferred_element_type=jnp.float32)


def _k8(hat1, stats_e, eemb_pad, lp1, lp2):
    blk = pl.BlockSpec((1000, D), lambda i: (i, 0))
    full = lambda shp: pl.BlockSpec(shp, lambda i: (0, 0))
    return pl.pallas_call(
        _k8_body,
        grid=(160,),
        in_specs=[blk, full((8 * NW, D)), full((8, D)), full((1, D)),
                  full((1, D)), full((D, D)), full((1, D))],
        out_specs=[blk, pl.BlockSpec((8, D), lambda i: (0, 0))],
        out_shape=[jax.ShapeDtypeStruct((E, D), jnp.float32),
                   jax.ShapeDtypeStruct((8, D), jnp.float32)],
    )(hat1, stats_e, eemb_pad,
      lp1["bn_e_g"].reshape(1, D), lp1["bn_e_b"].reshape(1, D),
      lp2["B3_w"], lp2["B3_b"].reshape(1, D))


def _k12b_body(a1_ref, g0_ref, g1_ref, h1_ref, st_ref, g_ref, bb_ref,
               w1, b1, w2, b2, w3, b3,
               y_ref, acc):
    i = pl.program_id(0)

    @pl.when(i == 0)
    def _():
        acc[...] = jnp.zeros_like(acc)

    xh = a1_ref[...] + g0_ref[...] + g1_ref[...]
    mu = st_ref[0:1, :] * (1.0 / N)
    var = st_ref[1:2, :] * (1.0 / N) - mu * mu
    hn = g_ref[...] * (xh - mu) * lax.rsqrt(var + 1e-5) + bb_ref[...]
    h2 = h1_ref[...] + jnp.maximum(hn, 0.0)
    acc[0:1, :] += jnp.sum(h2, axis=0, keepdims=True)

    @pl.when(i == pl.num_programs(0) - 1)
    def _():
        hg = acc[0:1, :] * (1.0 / N)
        y1 = jnp.maximum(_mm(hg, w1, b1), 0.0)
        y2 = jnp.maximum(_mm(y1, w2, b2), 0.0)
        y3 = lax.dot_general(y2, w3[...], (((1,), (1,)), ((), ())),
                             preferred_element_type=jnp.float32)
        y_ref[...] = y3[0:1, 0:1] + b3[0:1, 0:1]


def _k12b(a1h2, hg0, hg1, h1, stats, lp2, mlp):
    blk = pl.BlockSpec((1000, D), lambda i: (i, 0))
    full = lambda shp: pl.BlockSpec(shp, lambda i: (0, 0))
    (w1, b1), (w2, b2), (w3, b3) = mlp
    w1f = jnp.zeros((D, D), jnp.float32).at[:64, :].set(w1)
    b1p = jnp.zeros((1, D), jnp.float32).at[0, :64].set(b1)
    w2f = jnp.zeros((D, D), jnp.float32).at[:32, :64].set(w2)
    b2p = jnp.zeros((1, D), jnp.float32).at[0, :32].set(b2)
    w3f = jnp.zeros((8, D), jnp.float32).at[0:1, :32].set(w3)
    b3p = jnp.zeros((1, D), jnp.float32).at[0, 0].set(b3[0])
    return pl.pallas_call(
        _k12b_body,
        grid=(10,),
        in_specs=[blk, blk, blk, blk, full((8, D)), full((1, D)),
                  full((1, D)), full((D, D)), full((1, D)), full((D, D)),
                  full((1, D)), full((8, D)), full((1, D))],
        out_specs=pl.BlockSpec((1, 1), lambda i: (0, 0)),
        out_shape=jax.ShapeDtypeStruct((1, 1), jnp.float32),
        scratch_shapes=[pltpu.VMEM((8, D), jnp.float32)],
    )(a1h2, hg0, hg1, h1, stats,
      lp2["bn_h_g"].reshape(1, D), lp2["bn_h_b"].reshape(1, D),
      w1f, b1p, w2f, b2p, w3f, b3p)


def kernel(f, params, h, e, edge_index):
    lp1, lp2 = params["layers"]
    src = edge_index[0]
    dst = edge_index[1]
    eid = 4 * (jnp.arange(E, dtype=jnp.int32) % 128) + e

    f_i = _k1_findex(params["Wf"], f, params["bf"])

    pad = jnp.zeros((NPAD - N,), jnp.int32)
    h0, p0 = _k2_gather(params["node_embed"],
                        jnp.concatenate([h, pad]),
                        params["pos_embed"],
                        jnp.concatenate([f_i, pad]))

    eemb_pad = jnp.zeros((8, D), jnp.float32).at[:NBOND].set(
        params["edge_embed"])
    zer = jnp.zeros((RPT, D), jnp.float32)

    a1h1, b1h1, b2h1, c1p1, c2p1, a2v1, b3e1 = _k3_node_mats(
        h0, p0, eemb_pad, lp1)

    b3tab1 = jnp.tile(b3e1[:NBOND], (128, 1))
    hat1, ss0, ss1, stats_e = _sc_pass1(
        b1h1, b2h1, b3tab1, b3e1, src, dst, eid, zer,
        with_lin=False, with_stats=True)

    recip1 = _k5_recip(ss0, ss1)

    hagg1, pagg1 = _k6_pass2_l1(hat1, recip1, a2v1, c2p1, src, dst, zer)
    hagg1 = hagg1[:N]
    pagg1 = pagg1[:N]

    stats_h1 = _k_stats(a1h1, hagg1)
    h1, a1h2, b1h2, b2h2, a2v2 = _k7b(
        a1h1, hagg1, h0, c1p1, pagg1, p0, stats_h1, lp1, lp2)

    b3r, eb3 = _k8(hat1, stats_e, eemb_pad, lp1, lp2)

    b3tab2 = jnp.tile(eb3[:NBOND], (128, 1))
    sig2, ss0b, ss1b, _unused_stats = _sc_pass1(
        b1h2, b2h2, b3tab2, b3r, src, dst, eid, zer,
        with_lin=True, with_stats=False)

    recip2 = _k5_recip(ss0b, ss1b)

    hg2a, hg2b = _k11_pass2_l2(sig2, recip2, a2v2, src, dst, zer)
    hg2a = hg2a[:N]
    hg2b = hg2b[:N]

    stats_h2 = _k_stats(a1h2, hg2a, hg2b)
    y = _k12b(a1h2, hg2a, hg2b, h1, stats_h2, lp2, params["mlp"])
    return y


# speedup vs baseline: 2.3439x; 1.2144x over previous
import functools

import jax
import jax.numpy as jnp
from jax import lax
from jax.experimental import pallas as pl
from jax.experimental.pallas import tpu as pltpu
from jax.experimental.pallas import tpu_sc as plsc

N = 10000
E = 160000
D = 128
NBOND = 4

NC = 2
NS = 16
NW = NC * NS

C = 80
EPT_ALL = E // NW
EPT_CORE = E // NS
NP = 10240
RPT = NP // NS
GPT = 320
NPAD = GPT * NW

_mesh = functools.partial(
    plsc.VectorSubcoreMesh, core_axis_name="c", subcore_axis_name="s",
    num_cores=NC, num_subcores=NS)


def _sigmoid(x):
    return 1.0 / (1.0 + jnp.exp(-x))


def _mm(x, w_ref, b_ref):
    return lax.dot_general(x, w_ref[...], (((1,), (1,)), ((), ())),
                           preferred_element_type=jnp.float32) + b_ref[...]


def _matvec_body(wf_ref, f_ref, bf_ref, o_ref):
    r = lax.dot_general(
        wf_ref[...], f_ref[...], (((1,), (1,)), ((), ())),
        preferred_element_type=jnp.float32)
    v = r[:, 0:1] + bf_ref[:, 0:1]
    iv = jnp.clip(jnp.abs(v.astype(jnp.int32)), 0, N - 1)
    o_ref[...] = jnp.broadcast_to(iv, (400, 128))


def _k1_findex(Wf, f, bf):
    f2 = jnp.broadcast_to(f[None, :], (8, N))
    bf2 = jnp.broadcast_to(bf[:, None], (N, 128))
    out = pl.pallas_call(
        _matvec_body,
        grid=(25,),
        in_specs=[
            pl.BlockSpec((400, N), lambda i: (i, 0)),
            pl.BlockSpec((8, N), lambda i: (0, 0)),
            pl.BlockSpec((400, 128), lambda i: (i, 0)),
        ],
        out_specs=pl.BlockSpec((400, 128), lambda i: (i, 0)),
        out_shape=jax.ShapeDtypeStruct((N, 128), jnp.int32),
    )(Wf, f2, bf2)
    return out[:, 0]


def _k2_gather_body(ne_hbm, hidx_hbm, pe_hbm, fi_hbm, outh, outp,
                    idx_v, rows_v, sem):
    wid = lax.axis_index("s") * NC + lax.axis_index("c")
    base = wid * GPT
    pltpu.sync_copy(hidx_hbm.at[pl.ds(base, GPT)], idx_v)
    pltpu.async_copy(ne_hbm.at[idx_v], rows_v, sem).wait()
    pltpu.sync_copy(rows_v, outh.at[pl.ds(base, GPT)])
    pltpu.sync_copy(fi_hbm.at[pl.ds(base, GPT)], idx_v)
    pltpu.async_copy(pe_hbm.at[idx_v], rows_v, sem).wait()
    pltpu.sync_copy(rows_v, outp.at[pl.ds(base, GPT)])


def _k2_gather(node_embed, h_idx_pad, pos_embed, f_i_pad):
    fn = pl.kernel(
        _k2_gather_body,
        out_type=(jax.ShapeDtypeStruct((NPAD, D), jnp.float32),
                  jax.ShapeDtypeStruct((NPAD, D), jnp.float32)),
        mesh=_mesh(),
        scratch_types=[
            pltpu.VMEM((GPT,), jnp.int32),
            pltpu.VMEM((GPT, D), jnp.float32),
            pltpu.SemaphoreType.DMA,
        ],
    )
    outh, outp = fn(node_embed, h_idx_pad, pos_embed, f_i_pad)
    return outh[:N], outp[:N]


def _k3_body(h_ref, p_ref, eemb_ref,
             a1w, a1b, b1w, b1b, b2w, b2b, c1w, c1b, c2w, c2b,
             a2wh, a2wp, a2b, b3w, b3b,
             a1o, b1o, b2o, c1o, c2o, a2o, b3to):
    h = h_ref[...]
    p = p_ref[...]
    a1o[...] = _mm(h, a1w, a1b)
    b1o[...] = _mm(h, b1w, b1b)
    b2o[...] = _mm(h, b2w, b2b)
    c1o[...] = _mm(p, c1w, c1b)
    c2o[...] = _mm(p, c2w, c2b)
    a2o[...] = _mm(h, a2wh, a2b) + lax.dot_general(
        p, a2wp[...], (((1,), (1,)), ((), ())),
        preferred_element_type=jnp.float32)

    @pl.when(pl.program_id(0) == 0)
    def _():
        b3to[...] = _mm(eemb_ref[...], b3w, b3b)


def _k3_node_mats(h0, p0, eemb_pad, lp):
    full = lambda shp: pl.BlockSpec(shp, lambda i: (0, 0))
    blk = pl.BlockSpec((1000, D), lambda i: (i, 0))
    r1 = lambda b: b.reshape(1, D)
    return pl.pallas_call(
        _k3_body,
        grid=(10,),
        in_specs=[blk, blk, full((8, D))] + [full(s) for s in
                  [(D, D), (1, D), (D, D), (1, D), (D, D), (1, D),
                   (D, D), (1, D), (D, D), (1, D),
                   (D, D), (D, D), (1, D), (D, D), (1, D)]],
        out_specs=[blk] * 6 + [pl.BlockSpec((8, D), lambda i: (0, 0))],
        out_shape=[jax.ShapeDtypeStruct((N, D), jnp.float32)] * 6
        + [jax.ShapeDtypeStruct((8, D), jnp.float32)],
    )(h0, p0, eemb_pad,
      lp["A1_w"], r1(lp["A1_b"]), lp["B1_w"], r1(lp["B1_b"]),
      lp["B2_w"], r1(lp["B2_b"]), lp["C1_w"], r1(lp["C1_b"]),
      lp["C2_w"], r1(lp["C2_b"]),
      lp["A2_w"][:, :D], lp["A2_w"][:, D:], r1(lp["A2_b"]),
      lp["B3_w"], r1(lp["B3_b"]))


def _pass1_body(with_lin, with_stats,
                b1h_hbm, b2h_hbm, b3t_hbm, lin_hbm, src_hbm, dst_hbm,
                eid_hbm, zer_hbm,
                sigout_hbm, ss0_hbm, ss1_hbm, stats_hbm,
                sidx, didx, eidx, b1, b2, b3, b4, sbuf,
                shared, s1, s2, s3, s4):
    cid = lax.axis_index("c")
    sid = lax.axis_index("s")
    wid = sid * NC + cid
    pltpu.sync_copy(zer_hbm, shared.at[pl.ds(sid * RPT, RPT)])
    plsc.subcore_barrier()

    base = wid * EPT_ALL
    nch = EPT_ALL // C
    zv = jnp.zeros((16,), jnp.float32)
    stats0 = tuple(zv for _ in range(16))

    def chunk(k, stats):
        off = base + k * C
        pltpu.sync_copy(src_hbm.at[pl.ds(off, C)], sidx)
        pltpu.sync_copy(dst_hbm.at[pl.ds(off, C)], didx)
        pltpu.sync_copy(eid_hbm.at[pl.ds(off, C)], eidx)
        d1 = pltpu.async_copy(b1h_hbm.at[sidx], b1, s1)
        d2 = pltpu.async_copy(b2h_hbm.at[didx], b2, s2)
        d3 = pltpu.async_copy(b3t_hbm.at[eidx], b3, s3)
        if with_lin:
            d4 = pltpu.async_copy(lin_hbm.at[pl.ds(off, C)], b4, s4)
        d1.wait()
        d2.wait()
        d3.wait()
        if with_lin:
            d4.wait()

        def row(r, st):
            sums = []
            sqs = []
            for kk in range(8):
                sl = pl.ds(kk * 16, 16)
                v = b1[r, sl] + b2[r, sl] + b3[r, sl]
                if with_lin:
                    v = v + b4[r, sl]
                s = _sigmoid(v)
                b1[r, sl] = s
                if with_stats:
                    b3[r, sl] = v
                    sums.append(st[kk] + v)
                    sqs.append(st[8 + kk] + v * v)
            if with_stats:
                return tuple(sums) + tuple(sqs)
            return st

        stats = lax.fori_loop(0, C, row, stats)
        if with_stats:
            pltpu.sync_copy(b3, sigout_hbm.at[pl.ds(off, C)])
        else:
            pltpu.sync_copy(b1, sigout_hbm.at[pl.ds(off, C)])
        pltpu.sync_copy(b1, shared.at[didx], add=True)
        return stats

    stats = lax.fori_loop(0, nch, chunk, stats0)
    if with_stats:
        for kk in range(8):
            sbuf[0, pl.ds(kk * 16, 16)] = stats[kk]
            sbuf[1, pl.ds(kk * 16, 16)] = stats[8 + kk]
            for rr in range(2, 8):
                sbuf[rr, pl.ds(kk * 16, 16)] = zv
        pltpu.sync_copy(sbuf, stats_hbm.at[pl.ds(wid * 8, 8)])

    plsc.subcore_barrier()
    rows = pl.ds(sid * RPT, RPT)

    @pl.when(cid == 0)
    def _():
        pltpu.sync_copy(shared.at[rows], ss0_hbm.at[rows])

    @pl.when(cid == 1)
    def _():
        pltpu.sync_copy(shared.at[rows], ss1_hbm.at[rows])


def _sc_pass1(b1h, b2h, b3tab, lin, src, dst, eid, zer, with_lin, with_stats):
    body = functools.partial(_pass1_body, with_lin, with_stats)
    fn = pl.kernel(
        body,
        out_type=(jax.ShapeDtypeStruct((E, D), jnp.float32),
                  jax.ShapeDtypeStruct((NP, D), jnp.float32),
                  jax.ShapeDtypeStruct((NP, D), jnp.float32),
                  jax.ShapeDtypeStruct((NW * 8, D), jnp.float32)),
        mesh=_mesh(),
        scratch_types=[
            pltpu.VMEM((C,), jnp.int32),
            pltpu.VMEM((C,), jnp.int32),
            pltpu.VMEM((C,), jnp.int32),
            pltpu.VMEM((C, D), jnp.float32),
            pltpu.VMEM((C, D), jnp.float32),
            pltpu.VMEM((C, D), jnp.float32),
            pltpu.VMEM((C, D), jnp.float32),
            pltpu.VMEM((8, D), jnp.float32),
            pltpu.VMEM_SHARED((NP, D), jnp.float32),
            pltpu.SemaphoreType.DMA,
            pltpu.SemaphoreType.DMA,
            pltpu.SemaphoreType.DMA,
            pltpu.SemaphoreType.DMA,
        ],
    )
    return fn(b1h, b2h, b3tab, lin, src, dst, eid, zer)


def _recip_body(a_ref, b_ref, o_ref):
    o_ref[...] = 1.0 / (a_ref[...] + b_ref[...] + 1e-6)


def _k5_recip(ss0, ss1):
    blk = pl.BlockSpec((1024, D), lambda i: (i, 0))
    return pl.pallas_call(
        _recip_body, grid=(10,), in_specs=[blk, blk], out_specs=blk,
        out_shape=jax.ShapeDtypeStruct((NP, D), jnp.float32),
    )(ss0, ss1)


def _k6_body(hat_hbm, rec_hbm, a2v_hbm, c2p_hbm, src_hbm, dst_hbm, zer_hbm,
             hagg_hbm, pagg_hbm,
             sidx, didx, hb, rb, vb, shared, s1, s2, s3):
    cid = lax.axis_index("c")
    sid = lax.axis_index("s")
    pltpu.sync_copy(zer_hbm, shared.at[pl.ds(sid * RPT, RPT)])
    plsc.subcore_barrier()

    base = sid * EPT_CORE
    nch = EPT_CORE // C

    def chunk(k, carry):
        off = base + k * C
        pltpu.sync_copy(src_hbm.at[pl.ds(off, C)], sidx)
        pltpu.sync_copy(dst_hbm.at[pl.ds(off, C)], didx)
        d1 = pltpu.async_copy(hat_hbm.at[pl.ds(off, C)], hb, s1)
        d2 = pltpu.async_copy(rec_hbm.at[didx], rb, s2)

        @pl.when(cid == 0)
        def _():
            pltpu.async_copy(a2v_hbm.at[sidx], vb, s3).wait()

        @pl.when(cid == 1)
        def _():
            pltpu.async_copy(c2p_hbm.at[sidx], vb, s3).wait()
        d1.wait()
        d2.wait()

        def row(r, c):
            for kk in range(8):
                sl = pl.ds(kk * 16, 16)
                eta = _sigmoid(hb[r, sl]) * rb[r, sl]
                vb[r, sl] = eta * vb[r, sl]
            return c

        lax.fori_loop(0, C, row, 0)
        pltpu.sync_copy(vb, shared.at[didx], add=True)
        return carry

    lax.fori_loop(0, nch, chunk, 0)
    plsc.subcore_barrier()
    rows = pl.ds(sid * RPT, RPT)

    @pl.when(cid == 0)
    def _():
        pltpu.sync_copy(shared.at[rows], hagg_hbm.at[rows])

    @pl.when(cid == 1)
    def _():
        pltpu.sync_copy(shared.at[rows], pagg_hbm.at[rows])


def _k6_pass2_l1(hat1, recip1, a2v, c2p, src, dst, zer):
    fn = pl.kernel(
        _k6_body,
        out_type=(jax.ShapeDtypeStruct((NP, D), jnp.float32),
                  jax.ShapeDtypeStruct((NP, D), jnp.float32)),
        mesh=_mesh(),
        scratch_types=[
            pltpu.VMEM((C,), jnp.int32),
            pltpu.VMEM((C,), jnp.int32),
            pltpu.VMEM((C, D), jnp.float32),
            pltpu.VMEM((C, D), jnp.float32),
            pltpu.VMEM((C, D), jnp.float32),
            pltpu.VMEM_SHARED((NP, D), jnp.float32),
            pltpu.SemaphoreType.DMA,
            pltpu.SemaphoreType.DMA,
            pltpu.SemaphoreType.DMA,
        ],
    )
    return fn(hat1, recip1, a2v, c2p, src, dst, zer)


def _k11_body(sig_hbm, rec_hbm, a2v_hbm, src_hbm, dst_hbm, zer_hbm,
              h0_hbm, h1_hbm,
              sidx, didx, sb, rb, vb, shared, s1, s2, s3):
    cid = lax.axis_index("c")
    sid = lax.axis_index("s")
    wid = sid * NC + cid
    pltpu.sync_copy(zer_hbm, shared.at[pl.ds(sid * RPT, RPT)])
    plsc.subcore_barrier()

    base = wid * EPT_ALL
    nch = EPT_ALL // C

    def chunk(k, carry):
        off = base + k * C
        pltpu.sync_copy(src_hbm.at[pl.ds(off, C)], sidx)
        pltpu.sync_copy(dst_hbm.at[pl.ds(off, C)], didx)
        d1 = pltpu.async_copy(sig_hbm.at[pl.ds(off, C)], sb, s1)
        d2 = pltpu.async_copy(rec_hbm.at[didx], rb, s2)
        d3 = pltpu.async_copy(a2v_hbm.at[sidx], vb, s3)
        d1.wait()
        d2.wait()
        d3.wait()

        def row(r, c):
            for kk in range(8):
                sl = pl.ds(kk * 16, 16)
                vb[r, sl] = sb[r, sl] * rb[r, sl] * vb[r, sl]
            return c

        lax.fori_loop(0, C, row, 0)
        pltpu.sync_copy(vb, shared.at[didx], add=True)
        return carry

    lax.fori_loop(0, nch, chunk, 0)
    plsc.subcore_barrier()
    rows = pl.ds(sid * RPT, RPT)

    @pl.when(cid == 0)
    def _():
        pltpu.sync_copy(shared.at[rows], h0_hbm.at[rows])

    @pl.when(cid == 1)
    def _():
        pltpu.sync_copy(shared.at[rows], h1_hbm.at[rows])


def _k11_pass2_l2(sig2, recip2, a2v2, src, dst, zer):
    fn = pl.kernel(
        _k11_body,
        out_type=(jax.ShapeDtypeStruct((NP, D), jnp.float32),
                  jax.ShapeDtypeStruct((NP, D), jnp.float32)),
        mesh=_mesh(),
        scratch_types=[
            pltpu.VMEM((C,), jnp.int32),
            pltpu.VMEM((C,), jnp.int32),
            pltpu.VMEM((C, D), jnp.float32),
            pltpu.VMEM((C, D), jnp.float32),
            pltpu.VMEM((C, D), jnp.float32),
            pltpu.VMEM_SHARED((NP, D), jnp.float32),
            pltpu.SemaphoreType.DMA,
            pltpu.SemaphoreType.DMA,
            pltpu.SemaphoreType.DMA,
        ],
    )
    return fn(sig2, recip2, a2v2, src, dst, zer)


def _stats_body(nparts, *refs):
    in_refs = refs[:nparts]
    o_ref = refs[nparts]
    acc = refs[nparts + 1]
    i = pl.program_id(0)

    @pl.when(i == 0)
    def _():
        acc[...] = jnp.zeros_like(acc)

    x = in_refs[0][...]
    for r in in_refs[1:]:
        x = x + r[...]
    acc[0:1, :] += jnp.sum(x, axis=0, keepdims=True)
    acc[1:2, :] += jnp.sum(x * x, axis=0, keepdims=True)

    @pl.when(i == pl.num_programs(0) - 1)
    def _():
        o_ref[...] = acc[...]


def _k_stats(*parts):
    blk = pl.BlockSpec((1000, D), lambda i: (i, 0))
    body = functools.partial(_stats_body, len(parts))
    return pl.pallas_call(
        body, grid=(10,),
        in_specs=[blk] * len(parts),
        out_specs=pl.BlockSpec((8, D), lambda i: (0, 0)),
        out_shape=jax.ShapeDtypeStruct((8, D), jnp.float32),
        scratch_shapes=[pltpu.VMEM((8, D), jnp.float32)],
    )(*parts)


def _k7b_body(a1_ref, hagg_ref, h0_ref, c1_ref, pagg_ref, p0_ref, st_ref,
              g_ref, bb_ref,
              a1w, a1b, b1w, b1b, b2w, b2b, a2wh, a2wp, a2b,
              h1o, a1o, b1o, b2o, a2o):
    xh = a1_ref[...] + hagg_ref[...]
    mu = st_ref[0:1, :] * (1.0 / N)
    var = st_ref[1:2, :] * (1.0 / N) - mu * mu
    hn = g_ref[...] * (xh - mu) * lax.rsqrt(var + 1e-5) + bb_ref[...]
    h1 = h0_ref[...] + jnp.maximum(hn, 0.0)
    p1 = p0_ref[...] + jnp.tanh(c1_ref[...] + pagg_ref[...])
    h1o[...] = h1
    a1o[...] = _mm(h1, a1w, a1b)
    b1o[...] = _mm(h1, b1w, b1b)
    b2o[...] = _mm(h1, b2w, b2b)
    a2o[...] = _mm(h1, a2wh, a2b) + lax.dot_general(
        p1, a2wp[...], (((1,), (1,)), ((), ())),
        preferred_element_type=jnp.float32)


def _k7b(a1h1, hagg, h0, c1p1, pagg, p0, stats, lp1, lp2):
    blk = pl.BlockSpec((1000, D), lambda i: (i, 0))
    full = lambda shp: pl.BlockSpec(shp, lambda i: (0, 0))
    r1 = lambda b: b.reshape(1, D)
    return pl.pallas_call(
        _k7b_body,
        grid=(10,),
        in_specs=[blk, blk, blk, blk, blk, blk, full((8, D)),
                  full((1, D)), full((1, D))] + [full(s) for s in
                  [(D, D), (1, D), (D, D), (1, D), (D, D), (1, D),
                   (D, D), (D, D), (1, D)]],
        out_specs=[blk] * 5,
        out_shape=[jax.ShapeDtypeStruct((N, D), jnp.float32)] * 5,
    )(a1h1, hagg, h0, c1p1, pagg, p0, stats,
      lp1["bn_h_g"].reshape(1, D), lp1["bn_h_b"].reshape(1, D),
      lp2["A1_w"], r1(lp2["A1_b"]), lp2["B1_w"], r1(lp2["B1_b"]),
      lp2["B2_w"], r1(lp2["B2_b"]),
      lp2["A2_w"][:, :D], lp2["A2_w"][:, D:], r1(lp2["A2_b"]))


def _k8_body(hat_ref, st_ref, eemb_ref, g_ref, bb_ref, b3w, b3b,
             out_ref, eb3_ref):
    st = st_ref[...]
    tot = jnp.sum(st.reshape(NW, 8, D), axis=0)
    mu = tot[0:1, :] * (1.0 / E)
    var = tot[1:2, :] * (1.0 / E) - mu * mu
    x = g_ref[...] * (hat_ref[...] - mu) * lax.rsqrt(var + 1e-5) + bb_ref[...]
    x = jnp.maximum(x, 0.0)
    out_ref[...] = _mm(x, b3w, b3b)

    @pl.when(pl.program_id(0) == 0)
    def _():
        eb3_ref[...] = lax.dot_general(
            eemb_ref[...], b3w[...], (((1,), (1,)), ((), ())),
            preferred_element_type=jnp.float32)


def _k8(hat1, stats_e, eemb_pad, lp1, lp2):
    blk = pl.BlockSpec((1000, D), lambda i: (i, 0))
    full = lambda shp: pl.BlockSpec(shp, lambda i: (0, 0))
    return pl.pallas_call(
        _k8_body,
        grid=(160,),
        in_specs=[blk, full((8 * NW, D)), full((8, D)), full((1, D)),
                  full((1, D)), full((D, D)), full((1, D))],
        out_specs=[blk, pl.BlockSpec((8, D), lambda i: (0, 0))],
        out_shape=[jax.ShapeDtypeStruct((E, D), jnp.float32),
                   jax.ShapeDtypeStruct((8, D), jnp.float32)],
    )(hat1, stats_e, eemb_pad,
      lp1["bn_e_g"].reshape(1, D), lp1["bn_e_b"].reshape(1, D),
      lp2["B3_w"], lp2["B3_b"].reshape(1, D))


def _k12b_body(a1_ref, g0_ref, g1_ref, h1_ref, st_ref, g_ref, bb_ref,
               w1, b1, w2, b2, w3, b3,
               y_ref, acc):
    i = pl.program_id(0)

    @pl.when(i == 0)
    def _():
        acc[...] = jnp.zeros_like(acc)

    xh = a1_ref[...] + g0_ref[...] + g1_ref[...]
    mu = st_ref[0:1, :] * (1.0 / N)
    var = st_ref[1:2, :] * (1.0 / N) - mu * mu
    hn = g_ref[...] * (xh - mu) * lax.rsqrt(var + 1e-5) + bb_ref[...]
    h2 = h1_ref[...] + jnp.maximum(hn, 0.0)
    acc[0:1, :] += jnp.sum(h2, axis=0, keepdims=True)

    @pl.when(i == pl.num_programs(0) - 1)
    def _():
        hg = acc[0:1, :] * (1.0 / N)
        y1 = jnp.maximum(_mm(hg, w1, b1), 0.0)
        y2 = jnp.maximum(_mm(y1, w2, b2), 0.0)
        y3 = lax.dot_general(y2, w3[...], (((1,), (1,)), ((), ())),
                             preferred_element_type=jnp.float32)
        y_ref[...] = y3[0:1, 0:1] + b3[0:1, 0:1]


def _k12b(a1h2, hg0, hg1, h1, stats, lp2, mlp):
    blk = pl.BlockSpec((1000, D), lambda i: (i, 0))
    full = lambda shp: pl.BlockSpec(shp, lambda i: (0, 0))
    (w1, b1), (w2, b2), (w3, b3) = mlp
    w1f = jnp.zeros((D, D), jnp.float32).at[:64, :].set(w1)
    b1p = jnp.zeros((1, D), jnp.float32).at[0, :64].set(b1)
    w2f = jnp.zeros((D, D), jnp.float32).at[:32, :64].set(w2)
    b2p = jnp.zeros((1, D), jnp.float32).at[0, :32].set(b2)
    w3f = jnp.zeros((8, D), jnp.float32).at[0:1, :32].set(w3)
    b3p = jnp.zeros((1, D), jnp.float32).at[0, 0].set(b3[0])
    return pl.pallas_call(
        _k12b_body,
        grid=(10,),
        in_specs=[blk, blk, blk, blk, full((8, D)), full((1, D)),
                  full((1, D)), full((D, D)), full((1, D)), full((D, D)),
                  full((1, D)), full((8, D)), full((1, D))],
        out_specs=pl.BlockSpec((1, 1), lambda i: (0, 0)),
        out_shape=jax.ShapeDtypeStruct((1, 1), jnp.float32),
        scratch_shapes=[pltpu.VMEM((8, D), jnp.float32)],
    )(a1h2, hg0, hg1, h1, stats,
      lp2["bn_h_g"].reshape(1, D), lp2["bn_h_b"].reshape(1, D),
      w1f, b1p, w2f, b2p, w3f, b3p)


def kernel(f, params, h, e, edge_index):
    lp1, lp2 = params["layers"]
    src = edge_index[0]
    dst = edge_index[1]
    eid = 4 * (jnp.arange(E, dtype=jnp.int32) % 128) + e

    f_i = _k1_findex(params["Wf"], f, params["bf"])

    pad = jnp.zeros((NPAD - N,), jnp.int32)
    h0, p0 = _k2_gather(params["node_embed"],
                        jnp.concatenate([h, pad]),
                        params["pos_embed"],
                        jnp.concatenate([f_i, pad]))

    eemb_pad = jnp.zeros((8, D), jnp.float32).at[:NBOND].set(
        params["edge_embed"])
    zer = jnp.zeros((RPT, D), jnp.float32)

    a1h1, b1h1, b2h1, c1p1, c2p1, a2v1, b3e1 = _k3_node_mats(
        h0, p0, eemb_pad, lp1)

    b3tab1 = jnp.tile(b3e1[:NBOND], (128, 1))
    hat1, ss0, ss1, stats_e = _sc_pass1(
        b1h1, b2h1, b3tab1, b3e1, src, dst, eid, zer,
        with_lin=False, with_stats=True)

    recip1 = _k5_recip(ss0, ss1)

    hagg1, pagg1 = _k6_pass2_l1(hat1, recip1, a2v1, c2p1, src, dst, zer)
    hagg1 = hagg1[:N]
    pagg1 = pagg1[:N]

    stats_h1 = _k_stats(a1h1, hagg1)
    h1, a1h2, b1h2, b2h2, a2v2 = _k7b(
        a1h1, hagg1, h0, c1p1, pagg1, p0, stats_h1, lp1, lp2)

    b3r, eb3 = _k8(hat1, stats_e, eemb_pad, lp1, lp2)

    b3tab2 = jnp.tile(eb3[:NBOND], (128, 1))
    sig2, ss0b, ss1b, _unused_stats = _sc_pass1(
        b1h2, b2h2, b3tab2, b3r, src, dst, eid, zer,
        with_lin=True, with_stats=False)

    recip2 = _k5_recip(ss0b, ss1b)

    hg2a, hg2b = _k11_pass2_l2(sig2, recip2, a2v2, src, dst, zer)
    hg2a = hg2a[:N]
    hg2b = hg2b[:N]

    stats_h2 = _k_stats(a1h2, hg2a, hg2b)
    y = _k12b(a1h2, hg2a, hg2b, h1, stats_h2, lp2, params["mlp"])
    return y


# speedup vs baseline: 2.7661x; 1.1801x over previous
import functools

import jax
import jax.numpy as jnp
from jax import lax
from jax.experimental import pallas as pl
from jax.experimental.pallas import tpu as pltpu
from jax.experimental.pallas import tpu_sc as plsc

N = 10000
E = 160000
D = 128
NBOND = 4

NC = 2
NS = 16
NW = NC * NS

C = 80
EPT_ALL = E // NW
EPT_CORE = E // NS
NP = 10240
RPT = NP // NS
GPT = 320
NPAD = GPT * NW

_mesh = functools.partial(
    plsc.VectorSubcoreMesh, core_axis_name="c", subcore_axis_name="s",
    num_cores=NC, num_subcores=NS)


def _sigmoid(x):
    return 1.0 / (1.0 + jnp.exp(-x))


def _mm(x, w_ref, b_ref):
    return lax.dot_general(x, w_ref[...], (((1,), (1,)), ((), ())),
                           preferred_element_type=jnp.float32) + b_ref[...]


def _matvec_body(wf_ref, f_ref, bf_ref, o_ref):
    r = lax.dot_general(
        wf_ref[...], f_ref[...], (((1,), (1,)), ((), ())),
        preferred_element_type=jnp.float32)
    v = r[:, 0:1] + bf_ref[:, 0:1]
    iv = jnp.clip(jnp.abs(v.astype(jnp.int32)), 0, N - 1)
    o_ref[...] = jnp.broadcast_to(iv, (400, 128))


def _k1_findex(Wf, f, bf):
    f2 = jnp.broadcast_to(f[None, :], (8, N))
    bf2 = jnp.broadcast_to(bf[:, None], (N, 128))
    out = pl.pallas_call(
        _matvec_body,
        grid=(25,),
        in_specs=[
            pl.BlockSpec((400, N), lambda i: (i, 0)),
            pl.BlockSpec((8, N), lambda i: (0, 0)),
            pl.BlockSpec((400, 128), lambda i: (i, 0)),
        ],
        out_specs=pl.BlockSpec((400, 128), lambda i: (i, 0)),
        out_shape=jax.ShapeDtypeStruct((N, 128), jnp.int32),
    )(Wf, f2, bf2)
    return out[:, 0]


def _k2_gather_body(ne_hbm, hidx_hbm, pe_hbm, fi_hbm, outh, outp,
                    idx_v, rows_v, shared, sem):
    cid = lax.axis_index("c")
    sid = lax.axis_index("s")
    wid = sid * NC + cid
    base = wid * GPT
    pltpu.sync_copy(pe_hbm.at[pl.ds(sid * RPT, RPT)],
                    shared.at[pl.ds(sid * RPT, RPT)])
    pltpu.sync_copy(hidx_hbm.at[pl.ds(base, GPT)], idx_v)
    pltpu.async_copy(ne_hbm.at[idx_v], rows_v, sem).wait()
    pltpu.sync_copy(rows_v, outh.at[pl.ds(base, GPT)])
    plsc.subcore_barrier()
    pltpu.sync_copy(fi_hbm.at[pl.ds(base, GPT)], idx_v)
    pltpu.async_copy(shared.at[idx_v], rows_v, sem).wait()
    pltpu.sync_copy(rows_v, outp.at[pl.ds(base, GPT)])


def _k2_gather(node_embed, h_idx_pad, pos_embed_pad, f_i_pad):
    fn = pl.kernel(
        _k2_gather_body,
        out_type=(jax.ShapeDtypeStruct((NPAD, D), jnp.float32),
                  jax.ShapeDtypeStruct((NPAD, D), jnp.float32)),
        mesh=_mesh(),
        scratch_types=[
            pltpu.VMEM((GPT,), jnp.int32),
            pltpu.VMEM((GPT, D), jnp.float32),
            pltpu.VMEM_SHARED((NP, D), jnp.float32),
            pltpu.SemaphoreType.DMA,
        ],
    )
    outh, outp = fn(node_embed, h_idx_pad, pos_embed_pad, f_i_pad)
    return outh[:N], outp[:N]


def _k3_body(h_ref, p_ref, eemb_ref,
             a1w, a1b, b1w, b1b, b2w, b2b, c1w, c1b, c2w, c2b,
             a2wh, a2wp, a2b, b3w, b3b,
             a1o, b1o, b2o, c1o, c2o, a2o, b3to):
    h = h_ref[...]
    p = p_ref[...]
    a1o[...] = _mm(h, a1w, a1b)
    b1o[...] = _mm(h, b1w, b1b)
    b2o[...] = _mm(h, b2w, b2b)
    c1o[...] = _mm(p, c1w, c1b)
    c2o[...] = _mm(p, c2w, c2b)
    a2o[...] = _mm(h, a2wh, a2b) + lax.dot_general(
        p, a2wp[...], (((1,), (1,)), ((), ())),
        preferred_element_type=jnp.float32)

    @pl.when(pl.program_id(0) == 0)
    def _():
        b3to[...] = _mm(eemb_ref[...], b3w, b3b)


def _k3_node_mats(h0, p0, eemb_pad, lp):
    full = lambda shp: pl.BlockSpec(shp, lambda i: (0, 0))
    blk = pl.BlockSpec((1000, D), lambda i: (i, 0))
    r1 = lambda b: b.reshape(1, D)
    return pl.pallas_call(
        _k3_body,
        grid=(10,),
        in_specs=[blk, blk, full((8, D))] + [full(s) for s in
                  [(D, D), (1, D), (D, D), (1, D), (D, D), (1, D),
                   (D, D), (1, D), (D, D), (1, D),
                   (D, D), (D, D), (1, D), (D, D), (1, D)]],
        out_specs=[blk] * 6 + [pl.BlockSpec((8, D), lambda i: (0, 0))],
        out_shape=[jax.ShapeDtypeStruct((N, D), jnp.float32)] * 6
        + [jax.ShapeDtypeStruct((8, D), jnp.float32)],
    )(h0, p0, eemb_pad,
      lp["A1_w"], r1(lp["A1_b"]), lp["B1_w"], r1(lp["B1_b"]),
      lp["B2_w"], r1(lp["B2_b"]), lp["C1_w"], r1(lp["C1_b"]),
      lp["C2_w"], r1(lp["C2_b"]),
      lp["A2_w"][:, :D], lp["A2_w"][:, D:], r1(lp["A2_b"]),
      lp["B3_w"], r1(lp["B3_b"]))


def _pass1_body(with_lin, with_stats,
                b1h_hbm, b2h_hbm, b3t_hbm, lin_hbm, src_hbm, dst_hbm,
                eid_hbm, zer_hbm,
                sigout_hbm, ss0_hbm, ss1_hbm, stats_hbm,
                sidx, didx, eidx, b1, b2, b3, b4, sbuf,
                shared, s1, s2, s3, s4):
    cid = lax.axis_index("c")
    sid = lax.axis_index("s")
    wid = sid * NC + cid
    pltpu.sync_copy(zer_hbm, shared.at[pl.ds(sid * RPT, RPT)])
    plsc.subcore_barrier()

    base = wid * EPT_ALL
    nch = EPT_ALL // C
    zv = jnp.zeros((16,), jnp.float32)
    stats0 = tuple(zv for _ in range(16))

    def chunk(k, stats):
        off = base + k * C
        pltpu.sync_copy(src_hbm.at[pl.ds(off, C)], sidx)
        pltpu.sync_copy(dst_hbm.at[pl.ds(off, C)], didx)
        pltpu.sync_copy(eid_hbm.at[pl.ds(off, C)], eidx)
        d1 = pltpu.async_copy(b1h_hbm.at[sidx], b1, s1)
        d2 = pltpu.async_copy(b2h_hbm.at[didx], b2, s2)
        d3 = pltpu.async_copy(b3t_hbm.at[eidx], b3, s3)
        if with_lin:
            d4 = pltpu.async_copy(lin_hbm.at[pl.ds(off, C)], b4, s4)
        d1.wait()
        d2.wait()
        d3.wait()
        if with_lin:
            d4.wait()

        def row(r, st):
            sums = []
            sqs = []
            for kk in range(8):
                sl = pl.ds(kk * 16, 16)
                v = b1[r, sl] + b2[r, sl] + b3[r, sl]
                if with_lin:
                    v = v + b4[r, sl]
                s = _sigmoid(v)
                b1[r, sl] = s
                if with_stats:
                    b3[r, sl] = v
                    sums.append(st[kk] + v)
                    sqs.append(st[8 + kk] + v * v)
            if with_stats:
                return tuple(sums) + tuple(sqs)
            return st

        stats = lax.fori_loop(0, C, row, stats)
        if with_stats:
            pltpu.sync_copy(b3, sigout_hbm.at[pl.ds(off, C)])
        else:
            pltpu.sync_copy(b1, sigout_hbm.at[pl.ds(off, C)])
        pltpu.sync_copy(b1, shared.at[didx], add=True)
        return stats

    stats = lax.fori_loop(0, nch, chunk, stats0)
    if with_stats:
        for kk in range(8):
            sbuf[0, pl.ds(kk * 16, 16)] = stats[kk]
            sbuf[1, pl.ds(kk * 16, 16)] = stats[8 + kk]
            for rr in range(2, 8):
                sbuf[rr, pl.ds(kk * 16, 16)] = zv
        pltpu.sync_copy(sbuf, stats_hbm.at[pl.ds(wid * 8, 8)])

    plsc.subcore_barrier()
    rows = pl.ds(sid * RPT, RPT)

    @pl.when(cid == 0)
    def _():
        pltpu.sync_copy(shared.at[rows], ss0_hbm.at[rows])

    @pl.when(cid == 1)
    def _():
        pltpu.sync_copy(shared.at[rows], ss1_hbm.at[rows])


def _sc_pass1(b1h, b2h, b3tab, lin, src, dst, eid, zer, with_lin, with_stats):
    body = functools.partial(_pass1_body, with_lin, with_stats)
    fn = pl.kernel(
        body,
        out_type=(jax.ShapeDtypeStruct((E, D), jnp.float32),
                  jax.ShapeDtypeStruct((NP, D), jnp.float32),
                  jax.ShapeDtypeStruct((NP, D), jnp.float32),
                  jax.ShapeDtypeStruct((NW * 8, D), jnp.float32)),
        mesh=_mesh(),
        scratch_types=[
            pltpu.VMEM((C,), jnp.int32),
            pltpu.VMEM((C,), jnp.int32),
            pltpu.VMEM((C,), jnp.int32),
            pltpu.VMEM((C, D), jnp.float32),
            pltpu.VMEM((C, D), jnp.float32),
            pltpu.VMEM((C, D), jnp.float32),
            pltpu.VMEM((C, D), jnp.float32),
            pltpu.VMEM((8, D), jnp.float32),
            pltpu.VMEM_SHARED((NP, D), jnp.float32),
            pltpu.SemaphoreType.DMA,
            pltpu.SemaphoreType.DMA,
            pltpu.SemaphoreType.DMA,
            pltpu.SemaphoreType.DMA,
        ],
    )
    return fn(b1h, b2h, b3tab, lin, src, dst, eid, zer)


def _recip_body(a_ref, b_ref, o_ref):
    o_ref[...] = 1.0 / (a_ref[...] + b_ref[...] + 1e-6)


def _k5_recip(ss0, ss1):
    blk = pl.BlockSpec((1024, D), lambda i: (i, 0))
    return pl.pallas_call(
        _recip_body, grid=(10,), in_specs=[blk, blk], out_specs=blk,
        out_shape=jax.ShapeDtypeStruct((NP, D), jnp.float32),
    )(ss0, ss1)


def _k6_body(hat_hbm, rec_hbm, a2v_hbm, c2p_hbm, src_hbm, dst_hbm, zer_hbm,
             hagg_hbm, pagg_hbm,
             sidx, didx, hb, rb, vb, shared, s1, s2, s3):
    cid = lax.axis_index("c")
    sid = lax.axis_index("s")
    pltpu.sync_copy(zer_hbm, shared.at[pl.ds(sid * RPT, RPT)])
    plsc.subcore_barrier()

    base = sid * EPT_CORE
    nch = EPT_CORE // C

    def chunk(k, carry):
        off = base + k * C
        pltpu.sync_copy(src_hbm.at[pl.ds(off, C)], sidx)
        pltpu.sync_copy(dst_hbm.at[pl.ds(off, C)], didx)
        d1 = pltpu.async_copy(hat_hbm.at[pl.ds(off, C)], hb, s1)
        d2 = pltpu.async_copy(rec_hbm.at[didx], rb, s2)

        @pl.when(cid == 0)
        def _():
            pltpu.async_copy(a2v_hbm.at[sidx], vb, s3).wait()

        @pl.when(cid == 1)
        def _():
            pltpu.async_copy(c2p_hbm.at[sidx], vb, s3).wait()
        d1.wait()
        d2.wait()

        def row(r, c):
            for kk in range(8):
                sl = pl.ds(kk * 16, 16)
                eta = _sigmoid(hb[r, sl]) * rb[r, sl]
                vb[r, sl] = eta * vb[r, sl]
            return c

        lax.fori_loop(0, C, row, 0)
        pltpu.sync_copy(vb, shared.at[didx], add=True)
        return carry

    lax.fori_loop(0, nch, chunk, 0)
    plsc.subcore_barrier()
    rows = pl.ds(sid * RPT, RPT)

    @pl.when(cid == 0)
    def _():
        pltpu.sync_copy(shared.at[rows], hagg_hbm.at[rows])

    @pl.when(cid == 1)
    def _():
        pltpu.sync_copy(shared.at[rows], pagg_hbm.at[rows])


def _k6_pass2_l1(hat1, recip1, a2v, c2p, src, dst, zer):
    fn = pl.kernel(
        _k6_body,
        out_type=(jax.ShapeDtypeStruct((NP, D), jnp.float32),
                  jax.ShapeDtypeStruct((NP, D), jnp.float32)),
        mesh=_mesh(),
        scratch_types=[
            pltpu.VMEM((C,), jnp.int32),
            pltpu.VMEM((C,), jnp.int32),
            pltpu.VMEM((C, D), jnp.float32),
            pltpu.VMEM((C, D), jnp.float32),
            pltpu.VMEM((C, D), jnp.float32),
            pltpu.VMEM_SHARED((NP, D), jnp.float32),
            pltpu.SemaphoreType.DMA,
            pltpu.SemaphoreType.DMA,
            pltpu.SemaphoreType.DMA,
        ],
    )
    return fn(hat1, recip1, a2v, c2p, src, dst, zer)


def _k11_body(sig_hbm, rec_hbm, a2v_hbm, src_hbm, dst_hbm, zer_hbm,
              h0_hbm, h1_hbm,
              sidx, didx, sb, rb, vb, shared, s1, s2, s3):
    cid = lax.axis_index("c")
    sid = lax.axis_index("s")
    wid = sid * NC + cid
    pltpu.sync_copy(zer_hbm, shared.at[pl.ds(sid * RPT, RPT)])
    plsc.subcore_barrier()

    base = wid * EPT_ALL
    nch = EPT_ALL // C

    def chunk(k, carry):
        off = base + k * C
        pltpu.sync_copy(src_hbm.at[pl.ds(off, C)], sidx)
        pltpu.sync_copy(dst_hbm.at[pl.ds(off, C)], didx)
        d1 = pltpu.async_copy(sig_hbm.at[pl.ds(off, C)], sb, s1)
        d2 = pltpu.async_copy(rec_hbm.at[didx], rb, s2)
        d3 = pltpu.async_copy(a2v_hbm.at[sidx], vb, s3)
        d1.wait()
        d2.wait()
        d3.wait()

        def row(r, c):
            for kk in range(8):
                sl = pl.ds(kk * 16, 16)
                vb[r, sl] = sb[r, sl] * rb[r, sl] * vb[r, sl]
            return c

        lax.fori_loop(0, C, row, 0)
        pltpu.sync_copy(vb, shared.at[didx], add=True)
        return carry

    lax.fori_loop(0, nch, chunk, 0)
    plsc.subcore_barrier()
    rows = pl.ds(sid * RPT, RPT)

    @pl.when(cid == 0)
    def _():
        pltpu.sync_copy(shared.at[rows], h0_hbm.at[rows])

    @pl.when(cid == 1)
    def _():
        pltpu.sync_copy(shared.at[rows], h1_hbm.at[rows])


def _k11_pass2_l2(sig2, recip2, a2v2, src, dst, zer):
    fn = pl.kernel(
        _k11_body,
        out_type=(jax.ShapeDtypeStruct((NP, D), jnp.float32),
                  jax.ShapeDtypeStruct((NP, D), jnp.float32)),
        mesh=_mesh(),
        scratch_types=[
            pltpu.VMEM((C,), jnp.int32),
            pltpu.VMEM((C,), jnp.int32),
            pltpu.VMEM((C, D), jnp.float32),
            pltpu.VMEM((C, D), jnp.float32),
            pltpu.VMEM((C, D), jnp.float32),
            pltpu.VMEM_SHARED((NP, D), jnp.float32),
            pltpu.SemaphoreType.DMA,
            pltpu.SemaphoreType.DMA,
            pltpu.SemaphoreType.DMA,
        ],
    )
    return fn(sig2, recip2, a2v2, src, dst, zer)


def _stats_body(nparts, *refs):
    in_refs = refs[:nparts]
    o_ref = refs[nparts]
    acc = refs[nparts + 1]
    i = pl.program_id(0)

    @pl.when(i == 0)
    def _():
        acc[...] = jnp.zeros_like(acc)

    x = in_refs[0][...]
    for r in in_refs[1:]:
        x = x + r[...]
    acc[0:1, :] += jnp.sum(x, axis=0, keepdims=True)
    acc[1:2, :] += jnp.sum(x * x, axis=0, keepdims=True)

    @pl.when(i == pl.num_programs(0) - 1)
    def _():
        o_ref[...] = acc[...]


def _k_stats(*parts):
    blk = pl.BlockSpec((1000, D), lambda i: (i, 0))
    body = functools.partial(_stats_body, len(parts))
    return pl.pallas_call(
        body, grid=(10,),
        in_specs=[blk] * len(parts),
        out_specs=pl.BlockSpec((8, D), lambda i: (0, 0)),
        out_shape=jax.ShapeDtypeStruct((8, D), jnp.float32),
        scratch_shapes=[pltpu.VMEM((8, D), jnp.float32)],
    )(*parts)


def _k7b_body(a1_ref, hagg_ref, h0_ref, c1_ref, pagg_ref, p0_ref, st_ref,
              g_ref, bb_ref,
              a1w, a1b, b1w, b1b, b2w, b2b, a2wh, a2wp, a2b,
              h1o, a1o, b1o, b2o, a2o):
    xh = a1_ref[...] + hagg_ref[...]
    mu = st_ref[0:1, :] * (1.0 / N)
    var = st_ref[1:2, :] * (1.0 / N) - mu * mu
    hn = g_ref[...] * (xh - mu) * lax.rsqrt(var + 1e-5) + bb_ref[...]
    h1 = h0_ref[...] + jnp.maximum(hn, 0.0)
    p1 = p0_ref[...] + jnp.tanh(c1_ref[...] + pagg_ref[...])
    h1o[...] = h1
    a1o[...] = _mm(h1, a1w, a1b)
    b1o[...] = _mm(h1, b1w, b1b)
    b2o[...] = _mm(h1, b2w, b2b)
    a2o[...] = _mm(h1, a2wh, a2b) + lax.dot_general(
        p1, a2wp[...], (((1,), (1,)), ((), ())),
        preferred_element_type=jnp.float32)


def _k7b(a1h1, hagg, h0, c1p1, pagg, p0, stats, lp1, lp2):
    blk = pl.BlockSpec((1000, D), lambda i: (i, 0))
    full = lambda shp: pl.BlockSpec(shp, lambda i: (0, 0))
    r1 = lambda b: b.reshape(1, D)
    return pl.pallas_call(
        _k7b_body,
        grid=(10,),
        in_specs=[blk, blk, blk, blk, blk, blk, full((8, D)),
                  full((1, D)), full((1, D))] + [full(s) for s in
                  [(D, D), (1, D), (D, D), (1, D), (D, D), (1, D),
                   (D, D), (D, D), (1, D)]],
        out_specs=[blk] * 5,
        out_shape=[jax.ShapeDtypeStruct((N, D), jnp.float32)] * 5,
    )(a1h1, hagg, h0, c1p1, pagg, p0, stats,
      lp1["bn_h_g"].reshape(1, D), lp1["bn_h_b"].reshape(1, D),
      lp2["A1_w"], r1(lp2["A1_b"]), lp2["B1_w"], r1(lp2["B1_b"]),
      lp2["B2_w"], r1(lp2["B2_b"]),
      lp2["A2_w"][:, :D], lp2["A2_w"][:, D:], r1(lp2["A2_b"]))


def _k8_body(hat_ref, st_ref, eemb_ref, g_ref, bb_ref, b3w, b3b,
             out_ref, eb3_ref):
    st = st_ref[...]
    tot = jnp.sum(st.reshape(NW, 8, D), axis=0)
    mu = tot[0:1, :] * (1.0 / E)
    var = tot[1:2, :] * (1.0 / E) - mu * mu
    x = g_ref[...] * (hat_ref[...] - mu) * lax.rsqrt(var + 1e-5) + bb_ref[...]
    x = jnp.maximum(x, 0.0)
    out_ref[...] = _mm(x, b3w, b3b)

    @pl.when(pl.program_id(0) == 0)
    def _():
        eb3_ref[...] = lax.dot_general(
            eemb_ref[...], b3w[...], (((1,), (1,)), ((), ())),
            preferred_element_type=jnp.float32)


def _k8(hat1, stats_e, eemb_pad, lp1, lp2):
    blk = pl.BlockSpec((1000, D), lambda i: (i, 0))
    full = lambda shp: pl.BlockSpec(shp, lambda i: (0, 0))
    return pl.pallas_call(
        _k8_body,
        grid=(160,),
        in_specs=[blk, full((8 * NW, D)), full((8, D)), full((1, D)),
                  full((1, D)), full((D, D)), full((1, D))],
        out_specs=[blk, pl.BlockSpec((8, D), lambda i: (0, 0))],
        out_shape=[jax.ShapeDtypeStruct((E, D), jnp.float32),
                   jax.ShapeDtypeStruct((8, D), jnp.float32)],
    )(hat1, stats_e, eemb_pad,
      lp1["bn_e_g"].reshape(1, D), lp1["bn_e_b"].reshape(1, D),
      lp2["B3_w"], lp2["B3_b"].reshape(1, D))


def _k12b_body(a1_ref, g0_ref, g1_ref, h1_ref, st_ref, g_ref, bb_ref,
               w1, b1, w2, b2, w3, b3,
               y_ref, acc):
    i = pl.program_id(0)

    @pl.when(i == 0)
    def _():
        acc[...] = jnp.zeros_like(acc)

    xh = a1_ref[...] + g0_ref[...] + g1_ref[...]
    mu = st_ref[0:1, :] * (1.0 / N)
    var = st_ref[1:2, :] * (1.0 / N) - mu * mu
    hn = g_ref[...] * (xh - mu) * lax.rsqrt(var + 1e-5) + bb_ref[...]
    h2 = h1_ref[...] + jnp.maximum(hn, 0.0)
    acc[0:1, :] += jnp.sum(h2, axis=0, keepdims=True)

    @pl.when(i == pl.num_programs(0) - 1)
    def _():
        hg = acc[0:1, :] * (1.0 / N)
        y1 = jnp.maximum(_mm(hg, w1, b1), 0.0)
        y2 = jnp.maximum(_mm(y1, w2, b2), 0.0)
        y3 = lax.dot_general(y2, w3[...], (((1,), (1,)), ((), ())),
                             preferred_element_type=jnp.float32)
        y_ref[...] = y3[0:1, 0:1] + b3[0:1, 0:1]


def _k12b(a1h2, hg0, hg1, h1, stats, lp2, mlp):
    blk = pl.BlockSpec((1000, D), lambda i: (i, 0))
    full = lambda shp: pl.BlockSpec(shp, lambda i: (0, 0))
    (w1, b1), (w2, b2), (w3, b3) = mlp
    w1f = jnp.zeros((D, D), jnp.float32).at[:64, :].set(w1)
    b1p = jnp.zeros((1, D), jnp.float32).at[0, :64].set(b1)
    w2f = jnp.zeros((D, D), jnp.float32).at[:32, :64].set(w2)
    b2p = jnp.zeros((1, D), jnp.float32).at[0, :32].set(b2)
    w3f = jnp.zeros((8, D), jnp.float32).at[0:1, :32].set(w3)
    b3p = jnp.zeros((1, D), jnp.float32).at[0, 0].set(b3[0])
    return pl.pallas_call(
        _k12b_body,
        grid=(10,),
        in_specs=[blk, blk, blk, blk, full((8, D)), full((1, D)),
                  full((1, D)), full((D, D)), full((1, D)), full((D, D)),
                  full((1, D)), full((8, D)), full((1, D))],
        out_specs=pl.BlockSpec((1, 1), lambda i: (0, 0)),
        out_shape=jax.ShapeDtypeStruct((1, 1), jnp.float32),
        scratch_shapes=[pltpu.VMEM((8, D), jnp.float32)],
    )(a1h2, hg0, hg1, h1, stats,
      lp2["bn_h_g"].reshape(1, D), lp2["bn_h_b"].reshape(1, D),
      w1f, b1p, w2f, b2p, w3f, b3p)


def kernel(f, params, h, e, edge_index):
    lp1, lp2 = params["layers"]
    src = edge_index[0]
    dst = edge_index[1]
    eid = 4 * (jnp.arange(E, dtype=jnp.int32) % 128) + e

    f_i = _k1_findex(params["Wf"], f, params["bf"])

    pad = jnp.zeros((NPAD - N,), jnp.int32)
    pe_pad = jnp.zeros((NP, D), jnp.float32).at[:N].set(params["pos_embed"])
    h0, p0 = _k2_gather(params["node_embed"],
                        jnp.concatenate([h, pad]),
                        pe_pad,
                        jnp.concatenate([f_i, pad]))

    eemb_pad = jnp.zeros((8, D), jnp.float32).at[:NBOND].set(
        params["edge_embed"])
    zer = jnp.zeros((RPT, D), jnp.float32)

    a1h1, b1h1, b2h1, c1p1, c2p1, a2v1, b3e1 = _k3_node_mats(
        h0, p0, eemb_pad, lp1)

    b3tab1 = jnp.tile(b3e1[:NBOND], (128, 1))
    hat1, ss0, ss1, stats_e = _sc_pass1(
        b1h1, b2h1, b3tab1, b3e1, src, dst, eid, zer,
        with_lin=False, with_stats=True)

    recip1 = _k5_recip(ss0, ss1)

    hagg1, pagg1 = _k6_pass2_l1(hat1, recip1, a2v1, c2p1, src, dst, zer)
    hagg1 = hagg1[:N]
    pagg1 = pagg1[:N]

    stats_h1 = _k_stats(a1h1, hagg1)
    h1, a1h2, b1h2, b2h2, a2v2 = _k7b(
        a1h1, hagg1, h0, c1p1, pagg1, p0, stats_h1, lp1, lp2)

    b3r, eb3 = _k8(hat1, stats_e, eemb_pad, lp1, lp2)

    b3tab2 = jnp.tile(eb3[:NBOND], (128, 1))
    sig2, ss0b, ss1b, _unused_stats = _sc_pass1(
        b1h2, b2h2, b3tab2, b3r, src, dst, eid, zer,
        with_lin=True, with_stats=False)

    recip2 = _k5_recip(ss0b, ss1b)

    hg2a, hg2b = _k11_pass2_l2(sig2, recip2, a2v2, src, dst, zer)
    hg2a = hg2a[:N]
    hg2b = hg2b[:N]

    stats_h2 = _k_stats(a1h2, hg2a, hg2b)
    y = _k12b(a1h2, hg2a, hg2b, h1, stats_h2, lp2, params["mlp"])
    return y
